# baseline (speedup 1.0000x reference)
import functools

import jax
import jax.numpy as jnp
from jax import lax
from jax.experimental import pallas as pl
from jax.experimental.pallas import tpu as pltpu
from jax.experimental.pallas import tpu_sc as plsc

_NC = 2
_NS = 16
_EPS = 1e-5


def _sc_edge_agg(h, src, dst):
    N, D = h.shape
    E = src.shape[0]
    NW = _NC * _NS
    EPT = E // NW
    K = 125
    NCH = EPT // K
    GB = 20
    NGRP = NCH // GB
    assert EPT * NW == E and NCH * K == EPT and GB % 2 == 0 and NGRP * GB == NCH
    eidx = jnp.stack([src.reshape(NW, NGRP, GB, K),
                      dst.reshape(NW, NGRP, GB, K)], axis=3)
    zeros = jnp.zeros((N, D), jnp.float32)
    RPT = (N // _NS) & ~7
    REM = N - RPT * _NS
    assert REM % 8 == 0

    mesh = plsc.VectorSubcoreMesh(core_axis_name="c", subcore_axis_name="s")

    @functools.partial(
        pl.kernel,
        out_type=jax.ShapeDtypeStruct((_NC, N, D), jnp.float32),
        mesh=mesh,
        scratch_types=[
            pltpu.VMEM((2, GB, 2, K), jnp.int32),
            pltpu.VMEM((2, K, D), jnp.float32),
            pltpu.VMEM_SHARED((N, D), jnp.float32),
            pltpu.SemaphoreType.DMA,
            pltpu.SemaphoreType.DMA,
            pltpu.SemaphoreType.DMA,
            pltpu.SemaphoreType.DMA,
            pltpu.SemaphoreType.DMA,
            pltpu.SemaphoreType.DMA,
        ],
    )
    def agg_kernel(h_hbm, eidx_hbm, z_hbm, out_hbm, idxg, gbuf,
                   acc, gsem0, gsem1, isem0, isem1, ssem0, ssem1):
        c = lax.axis_index("c")
        s = lax.axis_index("s")
        w = c * _NS + s
        isems = (isem0, isem1)
        gsems = (gsem0, gsem1)
        ssems = (ssem0, ssem1)

        def load_group(g, bg):
            pltpu.make_async_copy(eidx_hbm.at[w].at[g], idxg.at[bg],
                                  isems[bg]).start()

        def wait_group(bg):
            pltpu.make_async_copy(eidx_hbm.at[w].at[0], idxg.at[bg],
                                  isems[bg]).wait()

        def start_gather(bg, jj, b):
            pltpu.make_async_copy(h_hbm.at[idxg.at[bg].at[jj].at[0]],
                                  gbuf.at[b], gsems[b]).start()

        def wait_gather(b):
            pltpu.make_async_copy(h_hbm.at[idxg.at[0].at[0].at[0]],
                                  gbuf.at[b], gsems[b]).wait()

        def start_scatter(bg, jj, b):
            pltpu.make_async_copy(gbuf.at[b], acc.at[idxg.at[bg].at[jj].at[1]],
                                  ssems[b]).start(add=True)

        def wait_scatter(b):
            pltpu.make_async_copy(gbuf.at[b], acc.at[idxg.at[0].at[0].at[1]],
                                  ssems[b]).wait()

        load_group(0, 0)
        load_group(1, 1)

        pltpu.sync_copy(z_hbm.at[pl.ds(s * RPT, RPT)],
                        acc.at[pl.ds(s * RPT, RPT)])

        @pl.when(s == 0)
        def _():
            pltpu.sync_copy(z_hbm.at[pl.ds(_NS * RPT, REM)],
                            acc.at[pl.ds(_NS * RPT, REM)])

        plsc.subcore_barrier()

        for g in range(NGRP):
            bg = g & 1
            wait_group(bg)
            start_gather(bg, 0, 0)
            start_gather(bg, 1, 1)

            @pl.loop(0, GB // 2)
            def _(ii):
                jj0 = 2 * ii
                wait_gather(0)
                start_scatter(bg, jj0, 0)
                wait_gather(1)
                start_scatter(bg, jj0 + 1, 1)

                @pl.when(jj0 + 2 < GB)
                def _():
                    wait_scatter(0)
                    start_gather(bg, jj0 + 2, 0)

                @pl.when(jj0 + 3 < GB)
                def _():
                    wait_scatter(1)
                    start_gather(bg, jj0 + 3, 1)

            wait_scatter(0)
            wait_scatter(1)
            if g + 2 < NGRP:
                load_group(g + 2, bg)

        plsc.subcore_barrier()

        pltpu.sync_copy(acc.at[pl.ds(s * RPT, RPT)],
                        out_hbm.at[c].at[pl.ds(s * RPT, RPT)])

        @pl.when(s == 0)
        def _():
            pltpu.sync_copy(acc.at[pl.ds(_NS * RPT, REM)],
                            out_hbm.at[c].at[pl.ds(_NS * RPT, REM)])

    return agg_kernel(h, eidx, zeros)


def _tc_layer(h, agg0, agg1, wa, ba, ga, bea, wb, bb, gb, beb):
    N, D = h.shape
    H = wa.shape[1]
    R = 1000
    NB = N // R
    assert NB * R == N

    def body(h_ref, a0_ref, a1_ref, wa_ref, ba_ref, ga_ref, bea_ref,
             wb_ref, bb_ref, gb_ref, beb_ref, out_ref,
             zbuf, s1, s2, t1, t2, sc1, sh1, sc2, sh2):
        p = pl.program_id(0)
        j = pl.program_id(1)

        @pl.when(p == 0)
        def _():
            a = h_ref[...] + a0_ref[...] + a1_ref[...]
            z = jnp.dot(a, wa_ref[...],
                        preferred_element_type=jnp.float32) + ba_ref[...]
            zbuf[pl.ds(j * R, R), :] = z
            cs = jnp.sum(z, axis=0, keepdims=True)
            cq = jnp.sum(z * z, axis=0, keepdims=True)

            @pl.when(j == 0)
            def _():
                s1[...] = cs
                s2[...] = cq

            @pl.when(j > 0)
            def _():
                s1[...] += cs
                s2[...] += cq

        @pl.when(p == 1)
        def _():
            @pl.when(j == 0)
            def _():
                mean = s1[...] * (1.0 / N)
                var = s2[...] * (1.0 / N) - mean * mean
                sc = ga_ref[...] * lax.rsqrt(var + _EPS)
                sc1[...] = sc
                sh1[...] = bea_ref[...] - mean * sc

            z = zbuf[pl.ds(j * R, R), :]
            y = jnp.maximum(z * sc1[...] + sh1[...], 0.0)
            w = jnp.dot(y, wb_ref[...],
                        preferred_element_type=jnp.float32) + bb_ref[...]
            zbuf[pl.ds(j * R, R), :] = w
            cs = jnp.sum(w, axis=0, keepdims=True)
            cq = jnp.sum(w * w, axis=0, keepdims=True)

            @pl.when(j == 0)
            def _():
                t1[...] = cs
                t2[...] = cq

            @pl.when(j > 0)
            def _():
                t1[...] += cs
                t2[...] += cq

        @pl.when(p == 2)
        def _():
            @pl.when(j == 0)
            def _():
                mean = t1[...] * (1.0 / N)
                var = t2[...] * (1.0 / N) - mean * mean
                sc = gb_ref[...] * lax.rsqrt(var + _EPS)
                sc2[...] = sc
                sh2[...] = beb_ref[...] - mean * sc

            w = zbuf[pl.ds(j * R, R), :]
            out_ref[...] = jnp.maximum(w * sc2[...] + sh2[...], 0.0)

    row_spec = pl.BlockSpec((R, D), lambda p, j: (j, 0))
    full_spec = pl.BlockSpec((D, H), lambda p, j: (0, 0))
    vec_spec = pl.BlockSpec((1, H), lambda p, j: (0, 0))

    return pl.pallas_call(
        body,
        grid=(3, NB),
        in_specs=[row_spec, row_spec, row_spec,
                  full_spec, vec_spec, vec_spec, vec_spec,
                  full_spec, vec_spec, vec_spec, vec_spec],
        out_specs=pl.BlockSpec((R, H), lambda p, j: (j, 0)),
        out_shape=jax.ShapeDtypeStruct((N, H), jnp.float32),
        scratch_shapes=[
            pltpu.VMEM((N, H), jnp.float32),
            pltpu.VMEM((1, H), jnp.float32), pltpu.VMEM((1, H), jnp.float32),
            pltpu.VMEM((1, H), jnp.float32), pltpu.VMEM((1, H), jnp.float32),
            pltpu.VMEM((1, H), jnp.float32), pltpu.VMEM((1, H), jnp.float32),
            pltpu.VMEM((1, H), jnp.float32), pltpu.VMEM((1, H), jnp.float32),
        ],
    )(h, agg0, agg1, wa, ba, ga, bea, wb, bb, gb, beb)


def _tc_readout(h1, h2, h3, batch3d, w1a, w1b, w1c, b1, w2p, b2p):
    N, H = h1.shape
    G = 128
    R = 1000
    NB = N // R
    OP = w2p.shape[1]

    def body(b_ref, h1_ref, h2_ref, h3_ref, w1a_ref, w1b_ref, w1c_ref,
             b1_ref, w2_ref, b2_ref, out_ref, p1, p2, p3):
        j = pl.program_id(0)
        seg = b_ref[0]
        gi = lax.broadcasted_iota(jnp.int32, (G, R), 0)
        oh = (seg == gi).astype(jnp.float32)
        c1 = jnp.dot(oh, h1_ref[...], preferred_element_type=jnp.float32)
        c2 = jnp.dot(oh, h2_ref[...], preferred_element_type=jnp.float32)
        c3 = jnp.dot(oh, h3_ref[...], preferred_element_type=jnp.float32)

        @pl.when(j == 0)
        def _():
            p1[...] = c1
            p2[...] = c2
            p3[...] = c3

        @pl.when(j > 0)
        def _():
            p1[...] += c1
            p2[...] += c2
            p3[...] += c3

        @pl.when(j == NB - 1)
        def _():
            z = (jnp.dot(p1[...], w1a_ref[...],
                         preferred_element_type=jnp.float32)
                 + jnp.dot(p2[...], w1b_ref[...],
                           preferred_element_type=jnp.float32)
                 + jnp.dot(p3[...], w1c_ref[...],
                           preferred_element_type=jnp.float32)
                 + b1_ref[...])
            y = jnp.maximum(z, 0.0)
            out_ref[...] = jnp.dot(
                y, w2_ref[...], preferred_element_type=jnp.float32) + b2_ref[...]

    row_spec = pl.BlockSpec((R, H), lambda j: (j, 0))

    return pl.pallas_call(
        body,
        grid=(NB,),
        in_specs=[pl.BlockSpec((1, 1, R), lambda j: (j, 0, 0)),
                  row_spec, row_spec, row_spec,
                  pl.BlockSpec((H, H), lambda j: (0, 0)),
                  pl.BlockSpec((H, H), lambda j: (0, 0)),
                  pl.BlockSpec((H, H), lambda j: (0, 0)),
                  pl.BlockSpec((1, H), lambda j: (0, 0)),
                  pl.BlockSpec((H, OP), lambda j: (0, 0)),
                  pl.BlockSpec((1, OP), lambda j: (0, 0))],
        out_specs=pl.BlockSpec((G, OP), lambda j: (0, 0)),
        out_shape=jax.ShapeDtypeStruct((G, OP), jnp.float32),
        scratch_shapes=[
            pltpu.VMEM((G, H), jnp.float32),
            pltpu.VMEM((G, H), jnp.float32),
            pltpu.VMEM((G, H), jnp.float32),
        ],
    )(batch3d, h1, h2, h3, w1a, w1b, w1c, b1, w2p, b2p)


@jax.jit
def kernel(x, edge_index, batch, params):
    src = edge_index[0]
    dst = edge_index[1]
    H = params["W0a"].shape[1]
    C = params["W_lin2"].shape[1]

    h = x
    hs = []
    for l in range(3):
        agg = _sc_edge_agg(h, src, dst)
        h = _tc_layer(
            h, agg[0], agg[1],
            params[f"W{l}a"], params[f"b{l}a"].reshape(1, H),
            params[f"g{l}a"].reshape(1, H), params[f"be{l}a"].reshape(1, H),
            params[f"W{l}b"], params[f"b{l}b"].reshape(1, H),
            params[f"g{l}b"].reshape(1, H), params[f"be{l}b"].reshape(1, H))
        hs.append(h)

    w1 = params["W_lin1"]
    w2p = jnp.pad(params["W_lin2"], ((0, 0), (0, 128 - C)))
    b2p = jnp.pad(params["b_lin2"], (0, 128 - C)).reshape(1, 128)
    out = _tc_readout(hs[0], hs[1], hs[2], batch.reshape(-1, 1, 1000),
                      w1[0:H], w1[H:2 * H], w1[2 * H:3 * H],
                      params["b_lin1"].reshape(1, H), w2p, b2p)
    return out[:, :C]


# speedup vs baseline: 1.1959x; 1.1959x over previous
import functools

import jax
import jax.numpy as jnp
from jax import lax
from jax.experimental import pallas as pl
from jax.experimental.pallas import tpu as pltpu
from jax.experimental.pallas import tpu_sc as plsc

_NC = 2
_NS = 16
_EPS = 1e-5


def _sc_edge_agg(h, src, dst):
    N, D = h.shape
    E = src.shape[0]
    NW = _NC * _NS
    EPT = E // NW
    K = 125
    NCH = EPT // K
    GB = 20
    NGRP = NCH // GB
    assert EPT * NW == E and NCH * K == EPT and GB % 2 == 0 and NGRP * GB == NCH
    eidx = jnp.stack([src.reshape(NW, NGRP, GB, K),
                      dst.reshape(NW, NGRP, GB, K)], axis=3)
    zeros = jnp.zeros((N, D), jnp.float32)
    RPT = (N // _NS) & ~7
    REM = N - RPT * _NS
    assert REM % 8 == 0

    mesh = plsc.VectorSubcoreMesh(core_axis_name="c", subcore_axis_name="s")

    @functools.partial(
        pl.kernel,
        out_type=jax.ShapeDtypeStruct((_NC, N, D), jnp.float32),
        mesh=mesh,
        scratch_types=[
            pltpu.VMEM((2, GB, 2, K), jnp.int32),
            pltpu.VMEM((2, K, D), jnp.float32),
            pltpu.VMEM_SHARED((N, D), jnp.float32),
            pltpu.SemaphoreType.DMA,
            pltpu.SemaphoreType.DMA,
            pltpu.SemaphoreType.DMA,
            pltpu.SemaphoreType.DMA,
        ],
    )
    def agg_kernel(h_hbm, eidx_hbm, z_hbm, out_hbm, idxg, gbuf,
                   acc, gsem0, gsem1, isem0, isem1):
        c = lax.axis_index("c")
        s = lax.axis_index("s")
        w = c * _NS + s
        isems = (isem0, isem1)
        gsems = (gsem0, gsem1)

        def load_group(g, bg):
            pltpu.make_async_copy(eidx_hbm.at[w].at[g], idxg.at[bg],
                                  isems[bg]).start()

        def wait_group(bg):
            pltpu.make_async_copy(eidx_hbm.at[w].at[0], idxg.at[bg],
                                  isems[bg]).wait()

        def start_gather(bg, jj, b):
            pltpu.make_async_copy(h_hbm.at[idxg.at[bg].at[jj].at[0]],
                                  gbuf.at[b], gsems[b]).start()

        def wait_gather(b):
            pltpu.make_async_copy(h_hbm.at[idxg.at[0].at[0].at[0]],
                                  gbuf.at[b], gsems[b]).wait()

        def scatter(bg, jj, b):
            pltpu.sync_copy(gbuf.at[b], acc.at[idxg.at[bg].at[jj].at[1]],
                            add=True)

        load_group(0, 0)
        load_group(1, 1)

        pltpu.sync_copy(z_hbm.at[pl.ds(s * RPT, RPT)],
                        acc.at[pl.ds(s * RPT, RPT)])

        @pl.when(s == 0)
        def _():
            pltpu.sync_copy(z_hbm.at[pl.ds(_NS * RPT, REM)],
                            acc.at[pl.ds(_NS * RPT, REM)])

        plsc.subcore_barrier()

        for g in range(NGRP):
            bg = g & 1
            wait_group(bg)
            start_gather(bg, 0, 0)
            start_gather(bg, 1, 1)

            @pl.loop(0, GB // 2)
            def _(ii):
                jj0 = 2 * ii
                wait_gather(0)
                scatter(bg, jj0, 0)

                @pl.when(jj0 + 2 < GB)
                def _():
                    start_gather(bg, jj0 + 2, 0)

                wait_gather(1)
                scatter(bg, jj0 + 1, 1)

                @pl.when(jj0 + 3 < GB)
                def _():
                    start_gather(bg, jj0 + 3, 1)

            if g + 2 < NGRP:
                load_group(g + 2, bg)

        plsc.subcore_barrier()

        pltpu.sync_copy(acc.at[pl.ds(s * RPT, RPT)],
                        out_hbm.at[c].at[pl.ds(s * RPT, RPT)])

        @pl.when(s == 0)
        def _():
            pltpu.sync_copy(acc.at[pl.ds(_NS * RPT, REM)],
                            out_hbm.at[c].at[pl.ds(_NS * RPT, REM)])

    return agg_kernel(h, eidx, zeros)


def _tc_layer(h, agg0, agg1, wa, ba, ga, bea, wb, bb, gb, beb):
    N, D = h.shape
    H = wa.shape[1]
    R = 1000
    NB = N // R
    assert NB * R == N

    def body(h_ref, a0_ref, a1_ref, wa_ref, ba_ref, ga_ref, bea_ref,
             wb_ref, bb_ref, gb_ref, beb_ref, out_ref,
             zbuf, s1, s2, t1, t2, sc1, sh1, sc2, sh2):
        p = pl.program_id(0)
        j = pl.program_id(1)

        @pl.when(p == 0)
        def _():
            a = h_ref[...] + a0_ref[...] + a1_ref[...]
            z = jnp.dot(a, wa_ref[...],
                        preferred_element_type=jnp.float32) + ba_ref[...]
            zbuf[pl.ds(j * R, R), :] = z
            cs = jnp.sum(z, axis=0, keepdims=True)
            cq = jnp.sum(z * z, axis=0, keepdims=True)

            @pl.when(j == 0)
            def _():
                s1[...] = cs
                s2[...] = cq

            @pl.when(j > 0)
            def _():
                s1[...] += cs
                s2[...] += cq

        @pl.when(p == 1)
        def _():
            @pl.when(j == 0)
            def _():
                mean = s1[...] * (1.0 / N)
                var = s2[...] * (1.0 / N) - mean * mean
                sc = ga_ref[...] * lax.rsqrt(var + _EPS)
                sc1[...] = sc
                sh1[...] = bea_ref[...] - mean * sc

            z = zbuf[pl.ds(j * R, R), :]
            y = jnp.maximum(z * sc1[...] + sh1[...], 0.0)
            w = jnp.dot(y, wb_ref[...],
                        preferred_element_type=jnp.float32) + bb_ref[...]
            zbuf[pl.ds(j * R, R), :] = w
            cs = jnp.sum(w, axis=0, keepdims=True)
            cq = jnp.sum(w * w, axis=0, keepdims=True)

            @pl.when(j == 0)
            def _():
                t1[...] = cs
                t2[...] = cq

            @pl.when(j > 0)
            def _():
                t1[...] += cs
                t2[...] += cq

        @pl.when(p == 2)
        def _():
            @pl.when(j == 0)
            def _():
                mean = t1[...] * (1.0 / N)
                var = t2[...] * (1.0 / N) - mean * mean
                sc = gb_ref[...] * lax.rsqrt(var + _EPS)
                sc2[...] = sc
                sh2[...] = beb_ref[...] - mean * sc

            w = zbuf[pl.ds(j * R, R), :]
            out_ref[...] = jnp.maximum(w * sc2[...] + sh2[...], 0.0)

    row_spec = pl.BlockSpec((R, D), lambda p, j: (j, 0))
    full_spec = pl.BlockSpec((D, H), lambda p, j: (0, 0))
    vec_spec = pl.BlockSpec((1, H), lambda p, j: (0, 0))

    return pl.pallas_call(
        body,
        grid=(3, NB),
        in_specs=[row_spec, row_spec, row_spec,
                  full_spec, vec_spec, vec_spec, vec_spec,
                  full_spec, vec_spec, vec_spec, vec_spec],
        out_specs=pl.BlockSpec((R, H), lambda p, j: (j, 0)),
        out_shape=jax.ShapeDtypeStruct((N, H), jnp.float32),
        scratch_shapes=[
            pltpu.VMEM((N, H), jnp.float32),
            pltpu.VMEM((1, H), jnp.float32), pltpu.VMEM((1, H), jnp.float32),
            pltpu.VMEM((1, H), jnp.float32), pltpu.VMEM((1, H), jnp.float32),
            pltpu.VMEM((1, H), jnp.float32), pltpu.VMEM((1, H), jnp.float32),
            pltpu.VMEM((1, H), jnp.float32), pltpu.VMEM((1, H), jnp.float32),
        ],
    )(h, agg0, agg1, wa, ba, ga, bea, wb, bb, gb, beb)


def _tc_readout(h1, h2, h3, batch3d, w1a, w1b, w1c, b1, w2p, b2p):
    N, H = h1.shape
    G = 128
    R = 1000
    NB = N // R
    OP = w2p.shape[1]

    def body(b_ref, h1_ref, h2_ref, h3_ref, w1a_ref, w1b_ref, w1c_ref,
             b1_ref, w2_ref, b2_ref, out_ref, p1, p2, p3):
        j = pl.program_id(0)
        seg = b_ref[0]
        gi = lax.broadcasted_iota(jnp.int32, (G, R), 0)
        oh = (seg == gi).astype(jnp.float32)
        c1 = jnp.dot(oh, h1_ref[...], preferred_element_type=jnp.float32)
        c2 = jnp.dot(oh, h2_ref[...], preferred_element_type=jnp.float32)
        c3 = jnp.dot(oh, h3_ref[...], preferred_element_type=jnp.float32)

        @pl.when(j == 0)
        def _():
            p1[...] = c1
            p2[...] = c2
            p3[...] = c3

        @pl.when(j > 0)
        def _():
            p1[...] += c1
            p2[...] += c2
            p3[...] += c3

        @pl.when(j == NB - 1)
        def _():
            z = (jnp.dot(p1[...], w1a_ref[...],
                         preferred_element_type=jnp.float32)
                 + jnp.dot(p2[...], w1b_ref[...],
                           preferred_element_type=jnp.float32)
                 + jnp.dot(p3[...], w1c_ref[...],
                           preferred_element_type=jnp.float32)
                 + b1_ref[...])
            y = jnp.maximum(z, 0.0)
            out_ref[...] = jnp.dot(
                y, w2_ref[...], preferred_element_type=jnp.float32) + b2_ref[...]

    row_spec = pl.BlockSpec((R, H), lambda j: (j, 0))

    return pl.pallas_call(
        body,
        grid=(NB,),
        in_specs=[pl.BlockSpec((1, 1, R), lambda j: (j, 0, 0)),
                  row_spec, row_spec, row_spec,
                  pl.BlockSpec((H, H), lambda j: (0, 0)),
                  pl.BlockSpec((H, H), lambda j: (0, 0)),
                  pl.BlockSpec((H, H), lambda j: (0, 0)),
                  pl.BlockSpec((1, H), lambda j: (0, 0)),
                  pl.BlockSpec((H, OP), lambda j: (0, 0)),
                  pl.BlockSpec((1, OP), lambda j: (0, 0))],
        out_specs=pl.BlockSpec((G, OP), lambda j: (0, 0)),
        out_shape=jax.ShapeDtypeStruct((G, OP), jnp.float32),
        scratch_shapes=[
            pltpu.VMEM((G, H), jnp.float32),
            pltpu.VMEM((G, H), jnp.float32),
            pltpu.VMEM((G, H), jnp.float32),
        ],
    )(batch3d, h1, h2, h3, w1a, w1b, w1c, b1, w2p, b2p)


@jax.jit
def kernel(x, edge_index, batch, params):
    src = edge_index[0]
    dst = edge_index[1]
    H = params["W0a"].shape[1]
    C = params["W_lin2"].shape[1]

    h = x
    hs = []
    for l in range(3):
        agg = _sc_edge_agg(h, src, dst)
        h = _tc_layer(
            h, agg[0], agg[1],
            params[f"W{l}a"], params[f"b{l}a"].reshape(1, H),
            params[f"g{l}a"].reshape(1, H), params[f"be{l}a"].reshape(1, H),
            params[f"W{l}b"], params[f"b{l}b"].reshape(1, H),
            params[f"g{l}b"].reshape(1, H), params[f"be{l}b"].reshape(1, H))
        hs.append(h)

    w1 = params["W_lin1"]
    w2p = jnp.pad(params["W_lin2"], ((0, 0), (0, 128 - C)))
    b2p = jnp.pad(params["b_lin2"], (0, 128 - C)).reshape(1, 128)
    out = _tc_readout(hs[0], hs[1], hs[2], batch.reshape(-1, 1, 1000),
                      w1[0:H], w1[H:2 * H], w1[2 * H:3 * H],
                      params["b_lin1"].reshape(1, H), w2p, b2p)
    return out[:, :C]


# speedup vs baseline: 1.3491x; 1.1281x over previous
import functools

import jax
import jax.numpy as jnp
from jax import lax
from jax.experimental import pallas as pl
from jax.experimental.pallas import tpu as pltpu
from jax.experimental.pallas import tpu_sc as plsc

_NC = 2
_NS = 16
_EPS = 1e-5


def _sc_edge_agg(h, src, dst):
    N, D = h.shape
    E = src.shape[0]
    NW = _NC * _NS
    EPT = E // NW
    K = 125
    NCH = EPT // K
    GB = 20
    NGRP = NCH // GB
    assert EPT * NW == E and NCH * K == EPT and GB % 2 == 0 and NGRP * GB == NCH
    eidx = jnp.stack([src.reshape(NW, NGRP, GB, K),
                      dst.reshape(NW, NGRP, GB, K)], axis=3)
    zeros = jnp.zeros((N, D), jnp.float32)
    RPT = (N // _NS) & ~7
    REM = N - RPT * _NS
    assert REM % 8 == 0

    mesh = plsc.VectorSubcoreMesh(core_axis_name="c", subcore_axis_name="s")

    @functools.partial(
        pl.kernel,
        out_type=jax.ShapeDtypeStruct((_NC, N, D), jnp.float32),
        mesh=mesh,
        scratch_types=[
            pltpu.VMEM((2, GB, 2, K), jnp.int32),
            pltpu.VMEM((2, K, D), jnp.float32),
            pltpu.VMEM_SHARED((N, D), jnp.float32),
            pltpu.SemaphoreType.DMA,
            pltpu.SemaphoreType.DMA,
            pltpu.SemaphoreType.DMA,
            pltpu.SemaphoreType.DMA,
        ],
    )
    def agg_kernel(h_hbm, eidx_hbm, z_hbm, out_hbm, idxg, gbuf,
                   acc, gsem0, gsem1, isem0, isem1):
        c = lax.axis_index("c")
        s = lax.axis_index("s")
        w = c * _NS + s
        isems = (isem0, isem1)
        gsems = (gsem0, gsem1)

        def load_group(g, bg):
            pltpu.make_async_copy(eidx_hbm.at[w].at[g], idxg.at[bg],
                                  isems[bg]).start()

        def wait_group(bg):
            pltpu.make_async_copy(eidx_hbm.at[w].at[0], idxg.at[bg],
                                  isems[bg]).wait()

        def start_gather(bg, jj, b):
            pltpu.make_async_copy(h_hbm.at[idxg.at[bg].at[jj].at[0]],
                                  gbuf.at[b], gsems[b]).start()

        def wait_gather(b):
            pltpu.make_async_copy(h_hbm.at[idxg.at[0].at[0].at[0]],
                                  gbuf.at[b], gsems[b]).wait()

        def scatter(bg, jj, b):
            pltpu.sync_copy(gbuf.at[b], acc.at[idxg.at[bg].at[jj].at[1]],
                            add=True)

        load_group(0, 0)
        load_group(1, 1)

        pltpu.sync_copy(z_hbm.at[pl.ds(s * RPT, RPT)],
                        acc.at[pl.ds(s * RPT, RPT)])

        @pl.when(s == 0)
        def _():
            pltpu.sync_copy(z_hbm.at[pl.ds(_NS * RPT, REM)],
                            acc.at[pl.ds(_NS * RPT, REM)])

        plsc.subcore_barrier()

        for g in range(NGRP):
            bg = g & 1
            wait_group(bg)
            start_gather(bg, 0, 0)
            start_gather(bg, 1, 1)

            @pl.loop(0, GB // 2)
            def _(ii):
                jj0 = 2 * ii
                wait_gather(0)
                scatter(bg, jj0, 0)

                @pl.when(jj0 + 2 < GB)
                def _():
                    start_gather(bg, jj0 + 2, 0)

                wait_gather(1)
                scatter(bg, jj0 + 1, 1)

                @pl.when(jj0 + 3 < GB)
                def _():
                    start_gather(bg, jj0 + 3, 1)

            if g + 2 < NGRP:
                load_group(g + 2, bg)

        plsc.subcore_barrier()

        pltpu.sync_copy(acc.at[pl.ds(s * RPT, RPT)],
                        out_hbm.at[c].at[pl.ds(s * RPT, RPT)])

        @pl.when(s == 0)
        def _():
            pltpu.sync_copy(acc.at[pl.ds(_NS * RPT, REM)],
                            out_hbm.at[c].at[pl.ds(_NS * RPT, REM)])

    return agg_kernel(h, eidx, zeros)


def _tc_layer(h, agg, wa, ba, ga, bea, wb, bb, gb, beb):
    N, D = h.shape
    H = wa.shape[1]
    R = 1000
    NB = N // R
    assert NB * R == N

    def body(h_ref, a_ref, wa_ref, ba_ref, ga_ref, bea_ref,
             wb_ref, bb_ref, gb_ref, beb_ref, out_ref,
             zbuf, s1, s2, t1, t2, sc1, sh1, sc2, sh2):
        p = pl.program_id(0)
        j = pl.program_id(1)

        @pl.when(p == 0)
        def _():
            a = h_ref[...] + a_ref[0] + a_ref[1]
            z = jnp.dot(a, wa_ref[...],
                        preferred_element_type=jnp.float32) + ba_ref[...]
            zbuf[pl.ds(j * R, R), :] = z
            cs = jnp.sum(z, axis=0, keepdims=True)
            cq = jnp.sum(z * z, axis=0, keepdims=True)

            @pl.when(j == 0)
            def _():
                s1[...] = cs
                s2[...] = cq

            @pl.when(j > 0)
            def _():
                s1[...] += cs
                s2[...] += cq

        @pl.when(p == 1)
        def _():
            @pl.when(j == 0)
            def _():
                mean = s1[...] * (1.0 / N)
                var = s2[...] * (1.0 / N) - mean * mean
                sc = ga_ref[...] * lax.rsqrt(var + _EPS)
                sc1[...] = sc
                sh1[...] = bea_ref[...] - mean * sc

            z = zbuf[pl.ds(j * R, R), :]
            y = jnp.maximum(z * sc1[...] + sh1[...], 0.0)
            w = jnp.dot(y, wb_ref[...],
                        preferred_element_type=jnp.float32) + bb_ref[...]
            zbuf[pl.ds(j * R, R), :] = w
            cs = jnp.sum(w, axis=0, keepdims=True)
            cq = jnp.sum(w * w, axis=0, keepdims=True)

            @pl.when(j == 0)
            def _():
                t1[...] = cs
                t2[...] = cq

            @pl.when(j > 0)
            def _():
                t1[...] += cs
                t2[...] += cq

        @pl.when(p == 2)
        def _():
            @pl.when(j == 0)
            def _():
                mean = t1[...] * (1.0 / N)
                var = t2[...] * (1.0 / N) - mean * mean
                sc = gb_ref[...] * lax.rsqrt(var + _EPS)
                sc2[...] = sc
                sh2[...] = beb_ref[...] - mean * sc

            w = zbuf[pl.ds(j * R, R), :]
            out_ref[...] = jnp.maximum(w * sc2[...] + sh2[...], 0.0)

    row_p0 = pl.BlockSpec((R, D), lambda p, j: (jnp.where(p == 0, j, 0), 0))
    agg_p0 = pl.BlockSpec((2, R, D),
                          lambda p, j: (0, jnp.where(p == 0, j, 0), 0))
    full_spec = pl.BlockSpec((D, H), lambda p, j: (0, 0))
    vec_spec = pl.BlockSpec((1, H), lambda p, j: (0, 0))

    return pl.pallas_call(
        body,
        grid=(3, NB),
        in_specs=[row_p0, agg_p0,
                  full_spec, vec_spec, vec_spec, vec_spec,
                  full_spec, vec_spec, vec_spec, vec_spec],
        out_specs=pl.BlockSpec((R, H),
                               lambda p, j: (jnp.where(p == 2, j, 0), 0)),
        out_shape=jax.ShapeDtypeStruct((N, H), jnp.float32),
        scratch_shapes=[
            pltpu.VMEM((N, H), jnp.float32),
            pltpu.VMEM((1, H), jnp.float32), pltpu.VMEM((1, H), jnp.float32),
            pltpu.VMEM((1, H), jnp.float32), pltpu.VMEM((1, H), jnp.float32),
            pltpu.VMEM((1, H), jnp.float32), pltpu.VMEM((1, H), jnp.float32),
            pltpu.VMEM((1, H), jnp.float32), pltpu.VMEM((1, H), jnp.float32),
        ],
    )(h, agg, wa, ba, ga, bea, wb, bb, gb, beb)


def _tc_pool(h, batch3d):
    N, H = h.shape
    G = 128
    R = 1000
    NB = N // R

    def body(b_ref, h_ref, out_ref, pacc):
        j = pl.program_id(0)
        seg = b_ref[0]
        gi = lax.broadcasted_iota(jnp.int32, (G, R), 0)
        oh = (seg == gi).astype(jnp.float32)
        c = jnp.dot(oh, h_ref[...], preferred_element_type=jnp.float32)

        @pl.when(j == 0)
        def _():
            pacc[...] = c

        @pl.when(j > 0)
        def _():
            pacc[...] += c

        @pl.when(j == NB - 1)
        def _():
            out_ref[...] = pacc[...]

    return pl.pallas_call(
        body,
        grid=(NB,),
        in_specs=[pl.BlockSpec((1, 1, R), lambda j: (j, 0, 0)),
                  pl.BlockSpec((R, H), lambda j: (j, 0))],
        out_specs=pl.BlockSpec((G, H), lambda j: (0, 0)),
        out_shape=jax.ShapeDtypeStruct((G, H), jnp.float32),
        scratch_shapes=[pltpu.VMEM((G, H), jnp.float32)],
    )(batch3d, h)


def _tc_pool3_head(h3, batch3d, pld1, pld2, w1a, w1b, w1c, b1, w2p, b2p):
    N, H = h3.shape
    G = 128
    R = 1000
    NB = N // R
    OP = w2p.shape[1]

    def body(b_ref, h3_ref, p1_ref, p2_ref, w1a_ref, w1b_ref, w1c_ref,
             b1_ref, w2_ref, b2_ref, out_ref, p3):
        j = pl.program_id(0)
        seg = b_ref[0]
        gi = lax.broadcasted_iota(jnp.int32, (G, R), 0)
        oh = (seg == gi).astype(jnp.float32)
        c3 = jnp.dot(oh, h3_ref[...], preferred_element_type=jnp.float32)

        @pl.when(j == 0)
        def _():
            p3[...] = c3

        @pl.when(j > 0)
        def _():
            p3[...] += c3

        @pl.when(j == NB - 1)
        def _():
            z = (jnp.dot(p1_ref[...], w1a_ref[...],
                         preferred_element_type=jnp.float32)
                 + jnp.dot(p2_ref[...], w1b_ref[...],
                           preferred_element_type=jnp.float32)
                 + jnp.dot(p3[...], w1c_ref[...],
                           preferred_element_type=jnp.float32)
                 + b1_ref[...])
            y = jnp.maximum(z, 0.0)
            out_ref[...] = jnp.dot(
                y, w2_ref[...], preferred_element_type=jnp.float32) + b2_ref[...]

    cst = lambda j: (0, 0)
    return pl.pallas_call(
        body,
        grid=(NB,),
        in_specs=[pl.BlockSpec((1, 1, R), lambda j: (j, 0, 0)),
                  pl.BlockSpec((R, H), lambda j: (j, 0)),
                  pl.BlockSpec((G, H), cst), pl.BlockSpec((G, H), cst),
                  pl.BlockSpec((H, H), cst), pl.BlockSpec((H, H), cst),
                  pl.BlockSpec((H, H), cst), pl.BlockSpec((1, H), cst),
                  pl.BlockSpec((H, OP), cst), pl.BlockSpec((1, OP), cst)],
        out_specs=pl.BlockSpec((G, OP), lambda j: (0, 0)),
        out_shape=jax.ShapeDtypeStruct((G, OP), jnp.float32),
        scratch_shapes=[pltpu.VMEM((G, H), jnp.float32)],
    )(batch3d, h3, pld1, pld2, w1a, w1b, w1c, b1, w2p, b2p)


@jax.jit
def kernel(x, edge_index, batch, params):
    src = edge_index[0]
    dst = edge_index[1]
    H = params["W0a"].shape[1]
    C = params["W_lin2"].shape[1]

    batch3d = batch.reshape(-1, 1, 1000)
    h = x
    hs = []
    for l in range(3):
        agg = _sc_edge_agg(h, src, dst)
        h = _tc_layer(
            h, agg,
            params[f"W{l}a"], params[f"b{l}a"].reshape(1, H),
            params[f"g{l}a"].reshape(1, H), params[f"be{l}a"].reshape(1, H),
            params[f"W{l}b"], params[f"b{l}b"].reshape(1, H),
            params[f"g{l}b"].reshape(1, H), params[f"be{l}b"].reshape(1, H))
        hs.append(h)

    pld1 = _tc_pool(hs[0], batch3d)
    pld2 = _tc_pool(hs[1], batch3d)

    w1 = params["W_lin1"]
    w2p = jnp.pad(params["W_lin2"], ((0, 0), (0, 128 - C)))
    b2p = jnp.pad(params["b_lin2"], (0, 128 - C)).reshape(1, 128)
    out = _tc_pool3_head(hs[2], batch3d, pld1, pld2,
                         w1[0:H], w1[H:2 * H], w1[2 * H:3 * H],
                         params["b_lin1"].reshape(1, H), w2p, b2p)
    return out[:, :C]


# speedup vs baseline: 1.4377x; 1.0657x over previous
import functools

import jax
import jax.numpy as jnp
from jax import lax
from jax.experimental import pallas as pl
from jax.experimental.pallas import tpu as pltpu
from jax.experimental.pallas import tpu_sc as plsc

_NC = 2
_NS = 16
_EPS = 1e-5


def _sc_edge_agg(h, src, dst):
    N, D = h.shape
    E = src.shape[0]
    NW = _NC * _NS
    EPT = E // NW
    K = 125
    NCH = EPT // K
    GB = 20
    NGRP = NCH // GB
    assert EPT * NW == E and NCH * K == EPT and GB % 2 == 0 and NGRP * GB == NCH
    eidx = jnp.stack([src.reshape(NW, NGRP, GB, K),
                      dst.reshape(NW, NGRP, GB, K)], axis=3)
    zeros = jnp.zeros((N, D), jnp.float32)
    RPT = (N // _NS) & ~7
    REM = N - RPT * _NS
    assert REM % 8 == 0

    mesh = plsc.VectorSubcoreMesh(core_axis_name="c", subcore_axis_name="s")

    @functools.partial(
        pl.kernel,
        out_type=jax.ShapeDtypeStruct((_NC, N, D), jnp.float32),
        mesh=mesh,
        scratch_types=[
            pltpu.VMEM((2, GB, 2, K), jnp.int32),
            pltpu.VMEM((2, K, D), jnp.float32),
            pltpu.VMEM_SHARED((N, D), jnp.float32),
            pltpu.SemaphoreType.DMA,
            pltpu.SemaphoreType.DMA,
            pltpu.SemaphoreType.DMA,
            pltpu.SemaphoreType.DMA,
            pltpu.SemaphoreType.DMA,
        ],
    )
    def agg_kernel(h_hbm, eidx_hbm, z_hbm, out_hbm, idxg, gbuf,
                   acc, gsem0, gsem1, isem0, isem1, zsem):
        c = lax.axis_index("c")
        s = lax.axis_index("s")
        w = c * _NS + s
        isems = (isem0, isem1)
        gsems = (gsem0, gsem1)

        def load_group(g, bg):
            pltpu.make_async_copy(eidx_hbm.at[w].at[g], idxg.at[bg],
                                  isems[bg]).start()

        def wait_group(bg):
            pltpu.make_async_copy(eidx_hbm.at[w].at[0], idxg.at[bg],
                                  isems[bg]).wait()

        def start_gather(bg, jj, b):
            pltpu.make_async_copy(h_hbm.at[idxg.at[bg].at[jj].at[0]],
                                  gbuf.at[b], gsems[b]).start()

        def wait_gather(b):
            pltpu.make_async_copy(h_hbm.at[idxg.at[0].at[0].at[0]],
                                  gbuf.at[b], gsems[b]).wait()

        def scatter(bg, jj, b):
            pltpu.sync_copy(gbuf.at[b], acc.at[idxg.at[bg].at[jj].at[1]],
                            add=True)

        def zero_main():
            return pltpu.make_async_copy(z_hbm.at[pl.ds(s * RPT, RPT)],
                                         acc.at[pl.ds(s * RPT, RPT)], zsem)

        def zero_rem():
            return pltpu.make_async_copy(z_hbm.at[pl.ds(_NS * RPT, REM)],
                                         acc.at[pl.ds(_NS * RPT, REM)], zsem)

        load_group(0, 0)
        load_group(1, 1)
        zero_main().start()

        @pl.when(s == 0)
        def _():
            zero_rem().start()

        wait_group(0)
        start_gather(0, 0, 0)
        start_gather(0, 1, 1)

        zero_main().wait()

        @pl.when(s == 0)
        def _():
            zero_rem().wait()

        plsc.subcore_barrier()

        for g in range(NGRP):
            bg = g & 1
            if g > 0:
                wait_group(bg)
                start_gather(bg, 0, 0)
                start_gather(bg, 1, 1)

            @pl.loop(0, GB // 2)
            def _(ii):
                jj0 = 2 * ii
                wait_gather(0)
                scatter(bg, jj0, 0)

                @pl.when(jj0 + 2 < GB)
                def _():
                    start_gather(bg, jj0 + 2, 0)

                wait_gather(1)
                scatter(bg, jj0 + 1, 1)

                @pl.when(jj0 + 3 < GB)
                def _():
                    start_gather(bg, jj0 + 3, 1)

            if g + 2 < NGRP:
                load_group(g + 2, bg)

        plsc.subcore_barrier()

        pltpu.sync_copy(acc.at[pl.ds(s * RPT, RPT)],
                        out_hbm.at[c].at[pl.ds(s * RPT, RPT)])

        @pl.when(s == 0)
        def _():
            pltpu.sync_copy(acc.at[pl.ds(_NS * RPT, REM)],
                            out_hbm.at[c].at[pl.ds(_NS * RPT, REM)])

    return agg_kernel(h, eidx, zeros)


def _tc_layer(h, agg, wa, ba, ga, bea, wb, bb, gb, beb):
    N, D = h.shape
    H = wa.shape[1]
    R = 2000
    NB = N // R
    assert NB * R == N

    def body(h_ref, a_ref, wa_ref, ba_ref, ga_ref, bea_ref,
             wb_ref, bb_ref, gb_ref, beb_ref, out_ref,
             zbuf, s1, s2, t1, t2, sc1, sh1, sc2, sh2):
        p = pl.program_id(0)
        j = pl.program_id(1)

        @pl.when(p == 0)
        def _():
            a = h_ref[...] + a_ref[0] + a_ref[1]
            z = jnp.dot(a, wa_ref[...],
                        preferred_element_type=jnp.float32) + ba_ref[...]
            zbuf[pl.ds(j * R, R), :] = z
            cs = jnp.sum(z, axis=0, keepdims=True)
            cq = jnp.sum(z * z, axis=0, keepdims=True)

            @pl.when(j == 0)
            def _():
                s1[...] = cs
                s2[...] = cq

            @pl.when(j > 0)
            def _():
                s1[...] += cs
                s2[...] += cq

        @pl.when(p == 1)
        def _():
            @pl.when(j == 0)
            def _():
                mean = s1[...] * (1.0 / N)
                var = s2[...] * (1.0 / N) - mean * mean
                sc = ga_ref[...] * lax.rsqrt(var + _EPS)
                sc1[...] = sc
                sh1[...] = bea_ref[...] - mean * sc

            z = zbuf[pl.ds(j * R, R), :]
            y = jnp.maximum(z * sc1[...] + sh1[...], 0.0)
            w = jnp.dot(y, wb_ref[...],
                        preferred_element_type=jnp.float32) + bb_ref[...]
            zbuf[pl.ds(j * R, R), :] = w
            cs = jnp.sum(w, axis=0, keepdims=True)
            cq = jnp.sum(w * w, axis=0, keepdims=True)

            @pl.when(j == 0)
            def _():
                t1[...] = cs
                t2[...] = cq

            @pl.when(j > 0)
            def _():
                t1[...] += cs
                t2[...] += cq

        @pl.when(p == 2)
        def _():
            @pl.when(j == 0)
            def _():
                mean = t1[...] * (1.0 / N)
                var = t2[...] * (1.0 / N) - mean * mean
                sc = gb_ref[...] * lax.rsqrt(var + _EPS)
                sc2[...] = sc
                sh2[...] = beb_ref[...] - mean * sc

            w = zbuf[pl.ds(j * R, R), :]
            out_ref[...] = jnp.maximum(w * sc2[...] + sh2[...], 0.0)

    row_p0 = pl.BlockSpec((R, D), lambda p, j: (jnp.where(p == 0, j, 0), 0))
    agg_p0 = pl.BlockSpec((2, R, D),
                          lambda p, j: (0, jnp.where(p == 0, j, 0), 0))
    full_spec = pl.BlockSpec((D, H), lambda p, j: (0, 0))
    vec_spec = pl.BlockSpec((1, H), lambda p, j: (0, 0))

    return pl.pallas_call(
        body,
        grid=(3, NB),
        in_specs=[row_p0, agg_p0,
                  full_spec, vec_spec, vec_spec, vec_spec,
                  full_spec, vec_spec, vec_spec, vec_spec],
        out_specs=pl.BlockSpec((R, H),
                               lambda p, j: (jnp.where(p == 2, j, 0), 0)),
        out_shape=jax.ShapeDtypeStruct((N, H), jnp.float32),
        scratch_shapes=[
            pltpu.VMEM((N, H), jnp.float32),
            pltpu.VMEM((1, H), jnp.float32), pltpu.VMEM((1, H), jnp.float32),
            pltpu.VMEM((1, H), jnp.float32), pltpu.VMEM((1, H), jnp.float32),
            pltpu.VMEM((1, H), jnp.float32), pltpu.VMEM((1, H), jnp.float32),
            pltpu.VMEM((1, H), jnp.float32), pltpu.VMEM((1, H), jnp.float32),
        ],
    )(h, agg, wa, ba, ga, bea, wb, bb, gb, beb)


def _tc_pool(h, batch3d):
    N, H = h.shape
    G = 128
    R = 2000
    NB = N // R

    def body(b_ref, h_ref, out_ref, pacc):
        j = pl.program_id(0)
        seg = b_ref[0]
        gi = lax.broadcasted_iota(jnp.int32, (G, R), 0)
        oh = (seg == gi).astype(jnp.float32)
        c = jnp.dot(oh, h_ref[...], preferred_element_type=jnp.float32)

        @pl.when(j == 0)
        def _():
            pacc[...] = c

        @pl.when(j > 0)
        def _():
            pacc[...] += c

        @pl.when(j == NB - 1)
        def _():
            out_ref[...] = pacc[...]

    return pl.pallas_call(
        body,
        grid=(NB,),
        in_specs=[pl.BlockSpec((1, 1, R), lambda j: (j, 0, 0)),
                  pl.BlockSpec((R, H), lambda j: (j, 0))],
        out_specs=pl.BlockSpec((G, H), lambda j: (0, 0)),
        out_shape=jax.ShapeDtypeStruct((G, H), jnp.float32),
        scratch_shapes=[pltpu.VMEM((G, H), jnp.float32)],
    )(batch3d, h)


def _tc_pool3_head(h3, batch3d, pld1, pld2, w1a, w1b, w1c, b1, w2p, b2p):
    N, H = h3.shape
    G = 128
    R = 2000
    NB = N // R
    OP = w2p.shape[1]

    def body(b_ref, h3_ref, p1_ref, p2_ref, w1a_ref, w1b_ref, w1c_ref,
             b1_ref, w2_ref, b2_ref, out_ref, p3):
        j = pl.program_id(0)
        seg = b_ref[0]
        gi = lax.broadcasted_iota(jnp.int32, (G, R), 0)
        oh = (seg == gi).astype(jnp.float32)
        c3 = jnp.dot(oh, h3_ref[...], preferred_element_type=jnp.float32)

        @pl.when(j == 0)
        def _():
            p3[...] = c3

        @pl.when(j > 0)
        def _():
            p3[...] += c3

        @pl.when(j == NB - 1)
        def _():
            z = (jnp.dot(p1_ref[...], w1a_ref[...],
                         preferred_element_type=jnp.float32)
                 + jnp.dot(p2_ref[...], w1b_ref[...],
                           preferred_element_type=jnp.float32)
                 + jnp.dot(p3[...], w1c_ref[...],
                           preferred_element_type=jnp.float32)
                 + b1_ref[...])
            y = jnp.maximum(z, 0.0)
            out_ref[...] = jnp.dot(
                y, w2_ref[...], preferred_element_type=jnp.float32) + b2_ref[...]

    cst = lambda j: (0, 0)
    return pl.pallas_call(
        body,
        grid=(NB,),
        in_specs=[pl.BlockSpec((1, 1, R), lambda j: (j, 0, 0)),
                  pl.BlockSpec((R, H), lambda j: (j, 0)),
                  pl.BlockSpec((G, H), cst), pl.BlockSpec((G, H), cst),
                  pl.BlockSpec((H, H), cst), pl.BlockSpec((H, H), cst),
                  pl.BlockSpec((H, H), cst), pl.BlockSpec((1, H), cst),
                  pl.BlockSpec((H, OP), cst), pl.BlockSpec((1, OP), cst)],
        out_specs=pl.BlockSpec((G, OP), lambda j: (0, 0)),
        out_shape=jax.ShapeDtypeStruct((G, OP), jnp.float32),
        scratch_shapes=[pltpu.VMEM((G, H), jnp.float32)],
    )(batch3d, h3, pld1, pld2, w1a, w1b, w1c, b1, w2p, b2p)


@jax.jit
def kernel(x, edge_index, batch, params):
    src = edge_index[0]
    dst = edge_index[1]
    H = params["W0a"].shape[1]
    C = params["W_lin2"].shape[1]

    batch3d = batch.reshape(-1, 1, 2000)
    h = x
    hs = []
    for l in range(3):
        agg = _sc_edge_agg(h, src, dst)
        h = _tc_layer(
            h, agg,
            params[f"W{l}a"], params[f"b{l}a"].reshape(1, H),
            params[f"g{l}a"].reshape(1, H), params[f"be{l}a"].reshape(1, H),
            params[f"W{l}b"], params[f"b{l}b"].reshape(1, H),
            params[f"g{l}b"].reshape(1, H), params[f"be{l}b"].reshape(1, H))
        hs.append(h)

    pld1 = _tc_pool(hs[0], batch3d)
    pld2 = _tc_pool(hs[1], batch3d)

    w1 = params["W_lin1"]
    w2p = jnp.pad(params["W_lin2"], ((0, 0), (0, 128 - C)))
    b2p = jnp.pad(params["b_lin2"], (0, 128 - C)).reshape(1, 128)
    out = _tc_pool3_head(hs[2], batch3d, pld1, pld2,
                         w1[0:H], w1[H:2 * H], w1[2 * H:3 * H],
                         params["b_lin1"].reshape(1, H), w2p, b2p)
    return out[:, :C]


# speedup vs baseline: 1.4678x; 1.0209x over previous
import functools

import jax
import jax.numpy as jnp
from jax import lax
from jax.experimental import pallas as pl
from jax.experimental.pallas import tpu as pltpu
from jax.experimental.pallas import tpu_sc as plsc

_NC = 2
_NS = 16
_EPS = 1e-5


def _sc_edge_agg(h, src, dst):
    N, D = h.shape
    E = src.shape[0]
    NW = _NC * _NS
    EPT = E // NW
    K = 125
    NCH = EPT // K
    GB = 20
    NGRP = NCH // GB
    assert EPT * NW == E and NCH * K == EPT and GB % 2 == 0 and NGRP * GB == NCH
    eidx = jnp.stack([src.reshape(NW, NGRP, GB, K),
                      dst.reshape(NW, NGRP, GB, K)], axis=3)
    zeros = jnp.zeros((N, D), jnp.float32)
    RPT = (N // _NS) & ~7
    REM = N - RPT * _NS
    assert REM % 8 == 0

    mesh = plsc.VectorSubcoreMesh(core_axis_name="c", subcore_axis_name="s")

    @functools.partial(
        pl.kernel,
        out_type=jax.ShapeDtypeStruct((_NC, N, D), jnp.float32),
        mesh=mesh,
        scratch_types=[
            pltpu.VMEM((2, GB, 2, K), jnp.int32),
            pltpu.VMEM((2, K, D), jnp.float32),
            pltpu.VMEM_SHARED((N, D), jnp.float32),
            pltpu.SemaphoreType.DMA,
            pltpu.SemaphoreType.DMA,
            pltpu.SemaphoreType.DMA,
            pltpu.SemaphoreType.DMA,
            pltpu.SemaphoreType.DMA,
        ],
    )
    def agg_kernel(h_hbm, eidx_hbm, z_hbm, out_hbm, idxg, gbuf,
                   acc, gsem0, gsem1, isem0, isem1, zsem):
        c = lax.axis_index("c")
        s = lax.axis_index("s")
        w = c * _NS + s
        isems = (isem0, isem1)
        gsems = (gsem0, gsem1)

        def load_group(g, bg):
            pltpu.make_async_copy(eidx_hbm.at[w].at[g], idxg.at[bg],
                                  isems[bg]).start()

        def wait_group(bg):
            pltpu.make_async_copy(eidx_hbm.at[w].at[0], idxg.at[bg],
                                  isems[bg]).wait()

        def start_gather(bg, jj, b):
            pltpu.make_async_copy(h_hbm.at[idxg.at[bg].at[jj].at[0]],
                                  gbuf.at[b], gsems[b]).start()

        def wait_gather(b):
            pltpu.make_async_copy(h_hbm.at[idxg.at[0].at[0].at[0]],
                                  gbuf.at[b], gsems[b]).wait()

        def scatter(bg, jj, b):
            pltpu.sync_copy(gbuf.at[b], acc.at[idxg.at[bg].at[jj].at[1]],
                            add=True)

        def init_main(ref):
            return pltpu.make_async_copy(ref.at[pl.ds(s * RPT, RPT)],
                                         acc.at[pl.ds(s * RPT, RPT)], zsem)

        def init_rem(ref):
            return pltpu.make_async_copy(ref.at[pl.ds(_NS * RPT, REM)],
                                         acc.at[pl.ds(_NS * RPT, REM)], zsem)

        load_group(0, 0)
        load_group(1, 1)

        @pl.when(c == 0)
        def _():
            init_main(h_hbm).start()

        @pl.when(c != 0)
        def _():
            init_main(z_hbm).start()

        @pl.when(jnp.logical_and(s == 0, c == 0))
        def _():
            init_rem(h_hbm).start()

        @pl.when(jnp.logical_and(s == 0, c != 0))
        def _():
            init_rem(z_hbm).start()

        wait_group(0)
        start_gather(0, 0, 0)
        start_gather(0, 1, 1)

        init_main(z_hbm).wait()

        @pl.when(s == 0)
        def _():
            init_rem(z_hbm).wait()

        plsc.subcore_barrier()

        for g in range(NGRP):
            bg = g & 1
            if g > 0:
                wait_group(bg)
                start_gather(bg, 0, 0)
                start_gather(bg, 1, 1)

            @pl.loop(0, GB // 2)
            def _(ii):
                jj0 = 2 * ii
                wait_gather(0)
                scatter(bg, jj0, 0)

                @pl.when(jj0 + 2 < GB)
                def _():
                    start_gather(bg, jj0 + 2, 0)

                wait_gather(1)
                scatter(bg, jj0 + 1, 1)

                @pl.when(jj0 + 3 < GB)
                def _():
                    start_gather(bg, jj0 + 3, 1)

            if g + 2 < NGRP:
                load_group(g + 2, bg)

        plsc.subcore_barrier()

        pltpu.sync_copy(acc.at[pl.ds(s * RPT, RPT)],
                        out_hbm.at[c].at[pl.ds(s * RPT, RPT)])

        @pl.when(s == 0)
        def _():
            pltpu.sync_copy(acc.at[pl.ds(_NS * RPT, REM)],
                            out_hbm.at[c].at[pl.ds(_NS * RPT, REM)])

    return agg_kernel(h, eidx, zeros)


def _tc_layer(agg, batch3d, wa, ba, ga, bea, wb, bb, gb, beb, head=None):
    _, N, D = agg.shape
    H = wa.shape[1]
    G = 128
    R = 2000
    NB = N // R
    assert NB * R == N

    def body(*refs):
        if head is None:
            (a_ref, b_ref, wa_ref, ba_ref, ga_ref, bea_ref,
             wb_ref, bb_ref, gb_ref, beb_ref,
             out_ref, pooled_ref,
             zbuf, s1, s2, t1, t2, sc1, sh1, sc2, sh2, pacc) = refs
        else:
            (a_ref, b_ref, wa_ref, ba_ref, ga_ref, bea_ref,
             wb_ref, bb_ref, gb_ref, beb_ref,
             p1_ref, p2_ref, w1a_ref, w1b_ref, w1c_ref, b1_ref,
             w2_ref, b2_ref,
             hout_ref,
             zbuf, s1, s2, t1, t2, sc1, sh1, sc2, sh2, pacc) = refs
        p = pl.program_id(0)
        j = pl.program_id(1)

        @pl.when(p == 0)
        def _():
            a = a_ref[0] + a_ref[1]
            z = jnp.dot(a, wa_ref[...],
                        preferred_element_type=jnp.float32) + ba_ref[...]
            zbuf[pl.ds(j * R, R), :] = z
            cs = jnp.sum(z, axis=0, keepdims=True)
            cq = jnp.sum(z * z, axis=0, keepdims=True)

            @pl.when(j == 0)
            def _():
                s1[...] = cs
                s2[...] = cq

            @pl.when(j > 0)
            def _():
                s1[...] += cs
                s2[...] += cq

        @pl.when(p == 1)
        def _():
            @pl.when(j == 0)
            def _():
                mean = s1[...] * (1.0 / N)
                var = s2[...] * (1.0 / N) - mean * mean
                sc = ga_ref[...] * lax.rsqrt(var + _EPS)
                sc1[...] = sc
                sh1[...] = bea_ref[...] - mean * sc

            z = zbuf[pl.ds(j * R, R), :]
            y = jnp.maximum(z * sc1[...] + sh1[...], 0.0)
            w = jnp.dot(y, wb_ref[...],
                        preferred_element_type=jnp.float32) + bb_ref[...]
            zbuf[pl.ds(j * R, R), :] = w
            cs = jnp.sum(w, axis=0, keepdims=True)
            cq = jnp.sum(w * w, axis=0, keepdims=True)

            @pl.when(j == 0)
            def _():
                t1[...] = cs
                t2[...] = cq

            @pl.when(j > 0)
            def _():
                t1[...] += cs
                t2[...] += cq

        @pl.when(p == 2)
        def _():
            @pl.when(j == 0)
            def _():
                mean = t1[...] * (1.0 / N)
                var = t2[...] * (1.0 / N) - mean * mean
                sc = gb_ref[...] * lax.rsqrt(var + _EPS)
                sc2[...] = sc
                sh2[...] = beb_ref[...] - mean * sc

            w = zbuf[pl.ds(j * R, R), :]
            y2 = jnp.maximum(w * sc2[...] + sh2[...], 0.0)
            if head is None:
                out_ref[...] = y2
            seg = b_ref[0]
            gi = lax.broadcasted_iota(jnp.int32, (G, R), 0)
            oh = (seg == gi).astype(jnp.float32)
            cp = jnp.dot(oh, y2, preferred_element_type=jnp.float32)

            @pl.when(j == 0)
            def _():
                pacc[...] = cp

            @pl.when(j > 0)
            def _():
                pacc[...] += cp

            @pl.when(j == NB - 1)
            def _():
                if head is None:
                    pooled_ref[...] = pacc[...]
                else:
                    z1 = (jnp.dot(p1_ref[...], w1a_ref[...],
                                  preferred_element_type=jnp.float32)
                          + jnp.dot(p2_ref[...], w1b_ref[...],
                                    preferred_element_type=jnp.float32)
                          + jnp.dot(pacc[...], w1c_ref[...],
                                    preferred_element_type=jnp.float32)
                          + b1_ref[...])
                    y1 = jnp.maximum(z1, 0.0)
                    hout_ref[...] = jnp.dot(
                        y1, w2_ref[...],
                        preferred_element_type=jnp.float32) + b2_ref[...]

    agg_p0 = pl.BlockSpec((2, R, D),
                          lambda p, j: (0, jnp.where(p == 0, j, 0), 0))
    b_p2 = pl.BlockSpec((1, 1, R),
                        lambda p, j: (jnp.where(p == 2, j, 0), 0, 0))
    full_spec = pl.BlockSpec((D, H), lambda p, j: (0, 0))
    vec_spec = pl.BlockSpec((1, H), lambda p, j: (0, 0))
    gh_spec = pl.BlockSpec((G, H), lambda p, j: (0, 0))

    in_specs = [agg_p0, b_p2,
                full_spec, vec_spec, vec_spec, vec_spec,
                full_spec, vec_spec, vec_spec, vec_spec]
    inputs = [agg, batch3d, wa, ba, ga, bea, wb, bb, gb, beb]
    if head is None:
        out_specs = [pl.BlockSpec((R, H),
                                  lambda p, j: (jnp.where(p == 2, j, 0), 0)),
                     gh_spec]
        out_shape = [jax.ShapeDtypeStruct((N, H), jnp.float32),
                     jax.ShapeDtypeStruct((G, H), jnp.float32)]
    else:
        pld1, pld2, w1a, w1b, w1c, b1, w2p, b2p = head
        OP = w2p.shape[1]
        in_specs += [gh_spec, gh_spec, full_spec, full_spec, full_spec,
                     vec_spec, pl.BlockSpec((H, OP), lambda p, j: (0, 0)),
                     pl.BlockSpec((1, OP), lambda p, j: (0, 0))]
        inputs += [pld1, pld2, w1a, w1b, w1c, b1, w2p, b2p]
        out_specs = pl.BlockSpec((G, OP), lambda p, j: (0, 0))
        out_shape = jax.ShapeDtypeStruct((G, OP), jnp.float32)

    return pl.pallas_call(
        body,
        grid=(3, NB),
        in_specs=in_specs,
        out_specs=out_specs,
        out_shape=out_shape,
        scratch_shapes=[
            pltpu.VMEM((N, H), jnp.float32),
            pltpu.VMEM((1, H), jnp.float32), pltpu.VMEM((1, H), jnp.float32),
            pltpu.VMEM((1, H), jnp.float32), pltpu.VMEM((1, H), jnp.float32),
            pltpu.VMEM((1, H), jnp.float32), pltpu.VMEM((1, H), jnp.float32),
            pltpu.VMEM((1, H), jnp.float32), pltpu.VMEM((1, H), jnp.float32),
            pltpu.VMEM((G, H), jnp.float32),
        ],
    )(*inputs)


@jax.jit
def kernel(x, edge_index, batch, params):
    src = edge_index[0]
    dst = edge_index[1]
    H = params["W0a"].shape[1]
    C = params["W_lin2"].shape[1]

    batch3d = batch.reshape(-1, 1, 2000)
    w1 = params["W_lin1"]
    w2p = jnp.pad(params["W_lin2"], ((0, 0), (0, 128 - C)))
    b2p = jnp.pad(params["b_lin2"], (0, 128 - C)).reshape(1, 128)

    def layer_params(l):
        return (params[f"W{l}a"], params[f"b{l}a"].reshape(1, H),
                params[f"g{l}a"].reshape(1, H), params[f"be{l}a"].reshape(1, H),
                params[f"W{l}b"], params[f"b{l}b"].reshape(1, H),
                params[f"g{l}b"].reshape(1, H), params[f"be{l}b"].reshape(1, H))

    h = x
    agg = _sc_edge_agg(h, src, dst)
    h, pld1 = _tc_layer(agg, batch3d, *layer_params(0))
    agg = _sc_edge_agg(h, src, dst)
    h, pld2 = _tc_layer(agg, batch3d, *layer_params(1))
    agg = _sc_edge_agg(h, src, dst)
    out = _tc_layer(agg, batch3d, *layer_params(2),
                    head=(pld1, pld2, w1[0:H], w1[H:2 * H], w1[2 * H:3 * H],
                          params["b_lin1"].reshape(1, H), w2p, b2p))
    return out[:, :C]


# speedup vs baseline: 1.4959x; 1.0192x over previous
import functools

import jax
import jax.numpy as jnp
from jax import lax
from jax.experimental import pallas as pl
from jax.experimental.pallas import tpu as pltpu
from jax.experimental.pallas import tpu_sc as plsc

_NC = 2
_NS = 16
_EPS = 1e-5


def _sc_edge_agg(h, src, dst):
    N, D = h.shape
    E = src.shape[0]
    NW = _NC * _NS
    EPT = E // NW
    K = 125
    NCH = EPT // K
    GB = 20
    NGRP = NCH // GB
    assert EPT * NW == E and NCH * K == EPT and GB % 2 == 0 and NGRP * GB == NCH
    eidx = jnp.stack([src.reshape(NW, NGRP, GB, K),
                      dst.reshape(NW, NGRP, GB, K)], axis=3)
    zeros = jnp.zeros((N, D), jnp.float32)
    RPT = (N // _NS) & ~7
    REM = N - RPT * _NS
    assert REM % 8 == 0

    mesh = plsc.VectorSubcoreMesh(core_axis_name="c", subcore_axis_name="s")

    @functools.partial(
        pl.kernel,
        out_type=jax.ShapeDtypeStruct((_NC, N, D), jnp.float32),
        mesh=mesh,
        scratch_types=[
            pltpu.VMEM((2, GB, 2, K), jnp.int32),
            pltpu.VMEM((2, K, D), jnp.float32),
            pltpu.VMEM_SHARED((N, D), jnp.float32),
            pltpu.SemaphoreType.DMA,
            pltpu.SemaphoreType.DMA,
            pltpu.SemaphoreType.DMA,
            pltpu.SemaphoreType.DMA,
            pltpu.SemaphoreType.DMA,
        ],
    )
    def agg_kernel(h_hbm, eidx_hbm, z_hbm, out_hbm, idxg, gbuf,
                   acc, gsem0, gsem1, isem0, isem1, zsem):
        c = lax.axis_index("c")
        s = lax.axis_index("s")
        w = c * _NS + s
        isems = (isem0, isem1)
        gsems = (gsem0, gsem1)

        def load_group(g, bg):
            pltpu.make_async_copy(eidx_hbm.at[w].at[g], idxg.at[bg],
                                  isems[bg]).start()

        def wait_group(bg):
            pltpu.make_async_copy(eidx_hbm.at[w].at[0], idxg.at[bg],
                                  isems[bg]).wait()

        def start_gather(bg, jj, b):
            pltpu.make_async_copy(h_hbm.at[idxg.at[bg].at[jj].at[0]],
                                  gbuf.at[b], gsems[b]).start()

        def wait_gather(b):
            pltpu.make_async_copy(h_hbm.at[idxg.at[0].at[0].at[0]],
                                  gbuf.at[b], gsems[b]).wait()

        def scatter(bg, jj, b):
            pltpu.sync_copy(gbuf.at[b], acc.at[idxg.at[bg].at[jj].at[1]],
                            add=True)

        def init_main(ref):
            return pltpu.make_async_copy(ref.at[pl.ds(s * RPT, RPT)],
                                         acc.at[pl.ds(s * RPT, RPT)], zsem)

        def init_rem(ref):
            return pltpu.make_async_copy(ref.at[pl.ds(_NS * RPT, REM)],
                                         acc.at[pl.ds(_NS * RPT, REM)], zsem)

        load_group(0, 0)
        load_group(1, 1)

        @pl.when(c == 0)
        def _():
            init_main(h_hbm).start()

        @pl.when(c != 0)
        def _():
            init_main(z_hbm).start()

        @pl.when(jnp.logical_and(s == 0, c == 0))
        def _():
            init_rem(h_hbm).start()

        @pl.when(jnp.logical_and(s == 0, c != 0))
        def _():
            init_rem(z_hbm).start()

        wait_group(0)
        start_gather(0, 0, 0)
        start_gather(0, 1, 1)

        init_main(z_hbm).wait()

        @pl.when(s == 0)
        def _():
            init_rem(z_hbm).wait()

        plsc.subcore_barrier()

        for g in range(NGRP):
            bg = g & 1
            if g > 0:
                wait_group(bg)
                start_gather(bg, 0, 0)
                start_gather(bg, 1, 1)

            @pl.loop(0, GB // 2)
            def _(ii):
                jj0 = 2 * ii
                wait_gather(0)
                scatter(bg, jj0, 0)

                @pl.when(jj0 + 2 < GB)
                def _():
                    start_gather(bg, jj0 + 2, 0)

                wait_gather(1)
                scatter(bg, jj0 + 1, 1)

                @pl.when(jj0 + 3 < GB)
                def _():
                    start_gather(bg, jj0 + 3, 1)

            if g + 2 < NGRP:
                load_group(g + 2, bg)

        plsc.subcore_barrier()

        pltpu.sync_copy(acc.at[pl.ds(s * RPT, RPT)],
                        out_hbm.at[c].at[pl.ds(s * RPT, RPT)])

        @pl.when(s == 0)
        def _():
            pltpu.sync_copy(acc.at[pl.ds(_NS * RPT, REM)],
                            out_hbm.at[c].at[pl.ds(_NS * RPT, REM)])

    return agg_kernel(h, eidx, zeros)


def _tc_layer(agg, batch3d, wa, ba, ga, bea, wb, bb, gb, beb, head=None):
    _, N, D = agg.shape
    H = wa.shape[1]
    G = 128
    R = 5000
    NB = N // R
    assert NB * R == N

    def body(*refs):
        if head is None:
            (a_ref, b_ref, wa_ref, ba_ref, ga_ref, bea_ref,
             wb_ref, bb_ref, gb_ref, beb_ref,
             out_ref, pooled_ref,
             zbuf, s1, s2, t1, t2, sc1, sh1, sc2, sh2, pacc) = refs
        else:
            (a_ref, b_ref, wa_ref, ba_ref, ga_ref, bea_ref,
             wb_ref, bb_ref, gb_ref, beb_ref,
             p1_ref, p2_ref, w1a_ref, w1b_ref, w1c_ref, b1_ref,
             w2_ref, b2_ref,
             hout_ref,
             zbuf, s1, s2, t1, t2, sc1, sh1, sc2, sh2, pacc) = refs
        p = pl.program_id(0)
        j = pl.program_id(1)

        @pl.when(p == 0)
        def _():
            a = a_ref[0] + a_ref[1]
            z = jnp.dot(a, wa_ref[...],
                        preferred_element_type=jnp.float32) + ba_ref[...]
            zbuf[pl.ds(j * R, R), :] = z
            cs = jnp.sum(z, axis=0, keepdims=True)
            cq = jnp.sum(z * z, axis=0, keepdims=True)

            @pl.when(j == 0)
            def _():
                s1[...] = cs
                s2[...] = cq

            @pl.when(j > 0)
            def _():
                s1[...] += cs
                s2[...] += cq

        @pl.when(p == 1)
        def _():
            @pl.when(j == 0)
            def _():
                mean = s1[...] * (1.0 / N)
                var = s2[...] * (1.0 / N) - mean * mean
                sc = ga_ref[...] * lax.rsqrt(var + _EPS)
                sc1[...] = sc
                sh1[...] = bea_ref[...] - mean * sc

            z = zbuf[pl.ds(j * R, R), :]
            y = jnp.maximum(z * sc1[...] + sh1[...], 0.0)
            w = jnp.dot(y, wb_ref[...],
                        preferred_element_type=jnp.float32) + bb_ref[...]
            zbuf[pl.ds(j * R, R), :] = w
            cs = jnp.sum(w, axis=0, keepdims=True)
            cq = jnp.sum(w * w, axis=0, keepdims=True)

            @pl.when(j == 0)
            def _():
                t1[...] = cs
                t2[...] = cq

            @pl.when(j > 0)
            def _():
                t1[...] += cs
                t2[...] += cq

        @pl.when(p == 2)
        def _():
            @pl.when(j == 0)
            def _():
                mean = t1[...] * (1.0 / N)
                var = t2[...] * (1.0 / N) - mean * mean
                sc = gb_ref[...] * lax.rsqrt(var + _EPS)
                sc2[...] = sc
                sh2[...] = beb_ref[...] - mean * sc

            w = zbuf[pl.ds(j * R, R), :]
            y2 = jnp.maximum(w * sc2[...] + sh2[...], 0.0)
            if head is None:
                out_ref[...] = y2
            seg = b_ref[0]
            gi = lax.broadcasted_iota(jnp.int32, (G, R), 0)
            oh = (seg == gi).astype(jnp.float32)
            cp = jnp.dot(oh, y2, preferred_element_type=jnp.float32)

            @pl.when(j == 0)
            def _():
                pacc[...] = cp

            @pl.when(j > 0)
            def _():
                pacc[...] += cp

            @pl.when(j == NB - 1)
            def _():
                if head is None:
                    pooled_ref[...] = pacc[...]
                else:
                    z1 = (jnp.dot(p1_ref[...], w1a_ref[...],
                                  preferred_element_type=jnp.float32)
                          + jnp.dot(p2_ref[...], w1b_ref[...],
                                    preferred_element_type=jnp.float32)
                          + jnp.dot(pacc[...], w1c_ref[...],
                                    preferred_element_type=jnp.float32)
                          + b1_ref[...])
                    y1 = jnp.maximum(z1, 0.0)
                    hout_ref[...] = jnp.dot(
                        y1, w2_ref[...],
                        preferred_element_type=jnp.float32) + b2_ref[...]

    agg_p0 = pl.BlockSpec((2, R, D),
                          lambda p, j: (0, jnp.where(p == 0, j, 0), 0))
    b_p2 = pl.BlockSpec((1, 1, R),
                        lambda p, j: (jnp.where(p == 2, j, 0), 0, 0))
    full_spec = pl.BlockSpec((D, H), lambda p, j: (0, 0))
    vec_spec = pl.BlockSpec((1, H), lambda p, j: (0, 0))
    gh_spec = pl.BlockSpec((G, H), lambda p, j: (0, 0))

    in_specs = [agg_p0, b_p2,
                full_spec, vec_spec, vec_spec, vec_spec,
                full_spec, vec_spec, vec_spec, vec_spec]
    inputs = [agg, batch3d, wa, ba, ga, bea, wb, bb, gb, beb]
    if head is None:
        out_specs = [pl.BlockSpec((R, H),
                                  lambda p, j: (jnp.where(p == 2, j, 0), 0)),
                     gh_spec]
        out_shape = [jax.ShapeDtypeStruct((N, H), jnp.float32),
                     jax.ShapeDtypeStruct((G, H), jnp.float32)]
    else:
        pld1, pld2, w1a, w1b, w1c, b1, w2p, b2p = head
        OP = w2p.shape[1]
        in_specs += [gh_spec, gh_spec, full_spec, full_spec, full_spec,
                     vec_spec, pl.BlockSpec((H, OP), lambda p, j: (0, 0)),
                     pl.BlockSpec((1, OP), lambda p, j: (0, 0))]
        inputs += [pld1, pld2, w1a, w1b, w1c, b1, w2p, b2p]
        out_specs = pl.BlockSpec((G, OP), lambda p, j: (0, 0))
        out_shape = jax.ShapeDtypeStruct((G, OP), jnp.float32)

    return pl.pallas_call(
        body,
        grid=(3, NB),
        in_specs=in_specs,
        out_specs=out_specs,
        out_shape=out_shape,
        scratch_shapes=[
            pltpu.VMEM((N, H), jnp.float32),
            pltpu.VMEM((1, H), jnp.float32), pltpu.VMEM((1, H), jnp.float32),
            pltpu.VMEM((1, H), jnp.float32), pltpu.VMEM((1, H), jnp.float32),
            pltpu.VMEM((1, H), jnp.float32), pltpu.VMEM((1, H), jnp.float32),
            pltpu.VMEM((1, H), jnp.float32), pltpu.VMEM((1, H), jnp.float32),
            pltpu.VMEM((G, H), jnp.float32),
        ],
    )(*inputs)


@jax.jit
def kernel(x, edge_index, batch, params):
    src = edge_index[0]
    dst = edge_index[1]
    H = params["W0a"].shape[1]
    C = params["W_lin2"].shape[1]

    batch3d = batch.reshape(-1, 1, 5000)
    w1 = params["W_lin1"]
    w2p = jnp.pad(params["W_lin2"], ((0, 0), (0, 128 - C)))
    b2p = jnp.pad(params["b_lin2"], (0, 128 - C)).reshape(1, 128)

    def layer_params(l):
        return (params[f"W{l}a"], params[f"b{l}a"].reshape(1, H),
                params[f"g{l}a"].reshape(1, H), params[f"be{l}a"].reshape(1, H),
                params[f"W{l}b"], params[f"b{l}b"].reshape(1, H),
                params[f"g{l}b"].reshape(1, H), params[f"be{l}b"].reshape(1, H))

    h = x
    agg = _sc_edge_agg(h, src, dst)
    h, pld1 = _tc_layer(agg, batch3d, *layer_params(0))
    agg = _sc_edge_agg(h, src, dst)
    h, pld2 = _tc_layer(agg, batch3d, *layer_params(1))
    agg = _sc_edge_agg(h, src, dst)
    out = _tc_layer(agg, batch3d, *layer_params(2),
                    head=(pld1, pld2, w1[0:H], w1[H:2 * H], w1[2 * H:3 * H],
                          params["b_lin1"].reshape(1, H), w2p, b2p))
    return out[:, :C]


# speedup vs baseline: 1.5429x; 1.0314x over previous
import functools

import jax
import jax.numpy as jnp
from jax import lax
from jax.experimental import pallas as pl
from jax.experimental.pallas import tpu as pltpu
from jax.experimental.pallas import tpu_sc as plsc

_NC = 2
_NS = 16
_EPS = 1e-5


def _sc_edge_agg(h, src, dst):
    N, D = h.shape
    E = src.shape[0]
    NW = _NC * _NS
    EPT = E // NW
    K = 125
    NCH = EPT // K
    GB = 20
    NGRP = NCH // GB
    assert EPT * NW == E and NCH * K == EPT and GB % 2 == 0 and NGRP * GB == NCH
    eidx = jnp.stack([src.reshape(NW, NGRP, GB, K),
                      dst.reshape(NW, NGRP, GB, K)], axis=3)
    zeros = jnp.zeros((N, D), jnp.float32)
    RPT = (N // _NS) & ~7
    REM = N - RPT * _NS
    assert REM % 8 == 0

    mesh = plsc.VectorSubcoreMesh(core_axis_name="c", subcore_axis_name="s")

    @functools.partial(
        pl.kernel,
        out_type=jax.ShapeDtypeStruct((_NC, N, D), jnp.float32),
        mesh=mesh,
        scratch_types=[
            pltpu.VMEM((2, GB, 2, K), jnp.int32),
            pltpu.VMEM((2, K, D), jnp.float32),
            pltpu.VMEM_SHARED((N, D), jnp.float32),
            pltpu.SemaphoreType.DMA,
            pltpu.SemaphoreType.DMA,
            pltpu.SemaphoreType.DMA,
            pltpu.SemaphoreType.DMA,
            pltpu.SemaphoreType.DMA,
        ],
    )
    def agg_kernel(h_hbm, eidx_hbm, z_hbm, out_hbm, idxg, gbuf,
                   acc, gsem0, gsem1, isem0, isem1, zsem):
        c = lax.axis_index("c")
        s = lax.axis_index("s")
        w = c * _NS + s
        isems = (isem0, isem1)
        gsems = (gsem0, gsem1)

        def load_group(g, bg):
            pltpu.make_async_copy(eidx_hbm.at[w].at[g], idxg.at[bg],
                                  isems[bg]).start()

        def wait_group(bg):
            pltpu.make_async_copy(eidx_hbm.at[w].at[0], idxg.at[bg],
                                  isems[bg]).wait()

        def start_gather(bg, jj, b):
            pltpu.make_async_copy(h_hbm.at[idxg.at[bg].at[jj].at[0]],
                                  gbuf.at[b], gsems[b]).start()

        def wait_gather(b):
            pltpu.make_async_copy(h_hbm.at[idxg.at[0].at[0].at[0]],
                                  gbuf.at[b], gsems[b]).wait()

        def scatter(bg, jj, b):
            pltpu.sync_copy(gbuf.at[b], acc.at[idxg.at[bg].at[jj].at[1]],
                            add=True)

        def init_main(ref):
            return pltpu.make_async_copy(ref.at[pl.ds(s * RPT, RPT)],
                                         acc.at[pl.ds(s * RPT, RPT)], zsem)

        def init_rem(ref):
            return pltpu.make_async_copy(ref.at[pl.ds(_NS * RPT, REM)],
                                         acc.at[pl.ds(_NS * RPT, REM)], zsem)

        load_group(0, 0)
        load_group(1, 1)

        @pl.when(c == 0)
        def _():
            init_main(h_hbm).start()

        @pl.when(c != 0)
        def _():
            init_main(z_hbm).start()

        @pl.when(jnp.logical_and(s == 0, c == 0))
        def _():
            init_rem(h_hbm).start()

        @pl.when(jnp.logical_and(s == 0, c != 0))
        def _():
            init_rem(z_hbm).start()

        wait_group(0)
        start_gather(0, 0, 0)
        start_gather(0, 1, 1)

        init_main(z_hbm).wait()

        @pl.when(s == 0)
        def _():
            init_rem(z_hbm).wait()

        plsc.subcore_barrier()

        for g in range(NGRP):
            bg = g & 1
            nbg = bg ^ 1

            @pl.loop(0, (GB - 2) // 2)
            def _(ii):
                jj0 = 2 * ii
                wait_gather(0)
                scatter(bg, jj0, 0)
                start_gather(bg, jj0 + 2, 0)
                wait_gather(1)
                scatter(bg, jj0 + 1, 1)
                start_gather(bg, jj0 + 3, 1)

            wait_gather(0)
            scatter(bg, GB - 2, 0)
            if g + 1 < NGRP:
                wait_group(nbg)
                start_gather(nbg, 0, 0)
            wait_gather(1)
            scatter(bg, GB - 1, 1)
            if g + 1 < NGRP:
                start_gather(nbg, 1, 1)
            if g + 2 < NGRP:
                load_group(g + 2, bg)

        plsc.subcore_barrier()

        pltpu.sync_copy(acc.at[pl.ds(s * RPT, RPT)],
                        out_hbm.at[c].at[pl.ds(s * RPT, RPT)])

        @pl.when(s == 0)
        def _():
            pltpu.sync_copy(acc.at[pl.ds(_NS * RPT, REM)],
                            out_hbm.at[c].at[pl.ds(_NS * RPT, REM)])

    return agg_kernel(h, eidx, zeros)


def _tc_layer(agg, batch3d, wa, ba, ga, bea, wb, bb, gb, beb, head=None):
    _, N, D = agg.shape
    H = wa.shape[1]
    G = 128
    R = 5000
    NB = N // R
    assert NB * R == N

    def body(*refs):
        if head is None:
            (a_ref, b_ref, wa_ref, ba_ref, ga_ref, bea_ref,
             wb_ref, bb_ref, gb_ref, beb_ref,
             out_ref, pooled_ref,
             zbuf, s1, s2, t1, t2, sc1, sh1, sc2, sh2, pacc) = refs
        else:
            (a_ref, b_ref, wa_ref, ba_ref, ga_ref, bea_ref,
             wb_ref, bb_ref, gb_ref, beb_ref,
             p1_ref, p2_ref, w1a_ref, w1b_ref, w1c_ref, b1_ref,
             w2_ref, b2_ref,
             hout_ref,
             zbuf, s1, s2, t1, t2, sc1, sh1, sc2, sh2, pacc) = refs
        p = pl.program_id(0)
        j = pl.program_id(1)

        @pl.when(p == 0)
        def _():
            a = a_ref[0] + a_ref[1]
            z = jnp.dot(a, wa_ref[...],
                        preferred_element_type=jnp.float32) + ba_ref[...]
            zbuf[pl.ds(j * R, R), :] = z
            cs = jnp.sum(z, axis=0, keepdims=True)
            cq = jnp.sum(z * z, axis=0, keepdims=True)

            @pl.when(j == 0)
            def _():
                s1[...] = cs
                s2[...] = cq

            @pl.when(j > 0)
            def _():
                s1[...] += cs
                s2[...] += cq

        @pl.when(p == 1)
        def _():
            @pl.when(j == 0)
            def _():
                mean = s1[...] * (1.0 / N)
                var = s2[...] * (1.0 / N) - mean * mean
                sc = ga_ref[...] * lax.rsqrt(var + _EPS)
                sc1[...] = sc
                sh1[...] = bea_ref[...] - mean * sc

            z = zbuf[pl.ds(j * R, R), :]
            y = jnp.maximum(z * sc1[...] + sh1[...], 0.0)
            w = jnp.dot(y, wb_ref[...],
                        preferred_element_type=jnp.float32) + bb_ref[...]
            zbuf[pl.ds(j * R, R), :] = w
            cs = jnp.sum(w, axis=0, keepdims=True)
            cq = jnp.sum(w * w, axis=0, keepdims=True)

            @pl.when(j == 0)
            def _():
                t1[...] = cs
                t2[...] = cq

            @pl.when(j > 0)
            def _():
                t1[...] += cs
                t2[...] += cq

        @pl.when(p == 2)
        def _():
            @pl.when(j == 0)
            def _():
                mean = t1[...] * (1.0 / N)
                var = t2[...] * (1.0 / N) - mean * mean
                sc = gb_ref[...] * lax.rsqrt(var + _EPS)
                sc2[...] = sc
                sh2[...] = beb_ref[...] - mean * sc

            w = zbuf[pl.ds(j * R, R), :]
            y2 = jnp.maximum(w * sc2[...] + sh2[...], 0.0)
            if head is None:
                out_ref[...] = y2
            seg = b_ref[0]
            gi = lax.broadcasted_iota(jnp.int32, (G, R), 0)
            oh = (seg == gi).astype(jnp.float32)
            cp = jnp.dot(oh, y2, preferred_element_type=jnp.float32)

            @pl.when(j == 0)
            def _():
                pacc[...] = cp

            @pl.when(j > 0)
            def _():
                pacc[...] += cp

            @pl.when(j == NB - 1)
            def _():
                if head is None:
                    pooled_ref[...] = pacc[...]
                else:
                    z1 = (jnp.dot(p1_ref[...], w1a_ref[...],
                                  preferred_element_type=jnp.float32)
                          + jnp.dot(p2_ref[...], w1b_ref[...],
                                    preferred_element_type=jnp.float32)
                          + jnp.dot(pacc[...], w1c_ref[...],
                                    preferred_element_type=jnp.float32)
                          + b1_ref[...])
                    y1 = jnp.maximum(z1, 0.0)
                    hout_ref[...] = jnp.dot(
                        y1, w2_ref[...],
                        preferred_element_type=jnp.float32) + b2_ref[...]

    agg_p0 = pl.BlockSpec((2, R, D),
                          lambda p, j: (0, jnp.where(p == 0, j, 0), 0))
    b_p2 = pl.BlockSpec((1, 1, R),
                        lambda p, j: (jnp.where(p == 2, j, 0), 0, 0))
    full_spec = pl.BlockSpec((D, H), lambda p, j: (0, 0))
    vec_spec = pl.BlockSpec((1, H), lambda p, j: (0, 0))
    gh_spec = pl.BlockSpec((G, H), lambda p, j: (0, 0))

    in_specs = [agg_p0, b_p2,
                full_spec, vec_spec, vec_spec, vec_spec,
                full_spec, vec_spec, vec_spec, vec_spec]
    inputs = [agg, batch3d, wa, ba, ga, bea, wb, bb, gb, beb]
    if head is None:
        out_specs = [pl.BlockSpec((R, H),
                                  lambda p, j: (jnp.where(p == 2, j, 0), 0)),
                     gh_spec]
        out_shape = [jax.ShapeDtypeStruct((N, H), jnp.float32),
                     jax.ShapeDtypeStruct((G, H), jnp.float32)]
    else:
        pld1, pld2, w1a, w1b, w1c, b1, w2p, b2p = head
        OP = w2p.shape[1]
        in_specs += [gh_spec, gh_spec, full_spec, full_spec, full_spec,
                     vec_spec, pl.BlockSpec((H, OP), lambda p, j: (0, 0)),
                     pl.BlockSpec((1, OP), lambda p, j: (0, 0))]
        inputs += [pld1, pld2, w1a, w1b, w1c, b1, w2p, b2p]
        out_specs = pl.BlockSpec((G, OP), lambda p, j: (0, 0))
        out_shape = jax.ShapeDtypeStruct((G, OP), jnp.float32)

    return pl.pallas_call(
        body,
        grid=(3, NB),
        in_specs=in_specs,
        out_specs=out_specs,
        out_shape=out_shape,
        scratch_shapes=[
            pltpu.VMEM((N, H), jnp.float32),
            pltpu.VMEM((1, H), jnp.float32), pltpu.VMEM((1, H), jnp.float32),
            pltpu.VMEM((1, H), jnp.float32), pltpu.VMEM((1, H), jnp.float32),
            pltpu.VMEM((1, H), jnp.float32), pltpu.VMEM((1, H), jnp.float32),
            pltpu.VMEM((1, H), jnp.float32), pltpu.VMEM((1, H), jnp.float32),
            pltpu.VMEM((G, H), jnp.float32),
        ],
    )(*inputs)


@jax.jit
def kernel(x, edge_index, batch, params):
    src = edge_index[0]
    dst = edge_index[1]
    H = params["W0a"].shape[1]
    C = params["W_lin2"].shape[1]

    batch3d = batch.reshape(-1, 1, 5000)
    w1 = params["W_lin1"]
    w2p = jnp.pad(params["W_lin2"], ((0, 0), (0, 128 - C)))
    b2p = jnp.pad(params["b_lin2"], (0, 128 - C)).reshape(1, 128)

    def layer_params(l):
        return (params[f"W{l}a"], params[f"b{l}a"].reshape(1, H),
                params[f"g{l}a"].reshape(1, H), params[f"be{l}a"].reshape(1, H),
                params[f"W{l}b"], params[f"b{l}b"].reshape(1, H),
                params[f"g{l}b"].reshape(1, H), params[f"be{l}b"].reshape(1, H))

    h = x
    agg = _sc_edge_agg(h, src, dst)
    h, pld1 = _tc_layer(agg, batch3d, *layer_params(0))
    agg = _sc_edge_agg(h, src, dst)
    h, pld2 = _tc_layer(agg, batch3d, *layer_params(1))
    agg = _sc_edge_agg(h, src, dst)
    out = _tc_layer(agg, batch3d, *layer_params(2),
                    head=(pld1, pld2, w1[0:H], w1[H:2 * H], w1[2 * H:3 * H],
                          params["b_lin1"].reshape(1, H), w2p, b2p))
    return out[:, :C]


# speedup vs baseline: 1.5591x; 1.0105x over previous
import functools

import jax
import jax.numpy as jnp
from jax import lax
from jax.experimental import pallas as pl
from jax.experimental.pallas import tpu as pltpu
from jax.experimental.pallas import tpu_sc as plsc

_NC = 2
_NS = 16
_EPS = 1e-5


def _sc_edge_agg(h, src, dst):
    N, D = h.shape
    E = src.shape[0]
    NW = _NC * _NS
    EPT = E // NW
    K = 125
    NCH = EPT // K
    GB = 20
    NGRP = NCH // GB
    assert EPT * NW == E and NCH * K == EPT and GB % 2 == 0 and NGRP * GB == NCH
    eidx = jnp.stack([src.reshape(NW, NGRP, GB, K),
                      dst.reshape(NW, NGRP, GB, K)], axis=3)
    zeros = jnp.zeros((N, D), jnp.float32)
    RPT = (N // _NS) & ~7
    REM = N - RPT * _NS
    assert REM % 8 == 0

    mesh = plsc.VectorSubcoreMesh(core_axis_name="c", subcore_axis_name="s")

    @functools.partial(
        pl.kernel,
        out_type=jax.ShapeDtypeStruct((_NC, N, D), jnp.float32),
        mesh=mesh,
        scratch_types=[
            pltpu.VMEM((2, GB, 2, K), jnp.int32),
            pltpu.VMEM((2, K, D), jnp.float32),
            pltpu.VMEM_SHARED((N, D), jnp.float32),
            pltpu.SemaphoreType.DMA,
            pltpu.SemaphoreType.DMA,
            pltpu.SemaphoreType.DMA,
            pltpu.SemaphoreType.DMA,
            pltpu.SemaphoreType.DMA,
        ],
    )
    def agg_kernel(h_hbm, eidx_hbm, z_hbm, out_hbm, idxg, gbuf,
                   acc, gsem0, gsem1, isem0, isem1, zsem):
        c = lax.axis_index("c")
        s = lax.axis_index("s")
        w = c * _NS + s
        isems = (isem0, isem1)
        gsems = (gsem0, gsem1)

        def load_group(g, bg):
            pltpu.make_async_copy(eidx_hbm.at[w].at[g], idxg.at[bg],
                                  isems[bg]).start()

        def wait_group(bg):
            pltpu.make_async_copy(eidx_hbm.at[w].at[0], idxg.at[bg],
                                  isems[bg]).wait()

        def start_gather(bg, jj, b):
            pltpu.make_async_copy(h_hbm.at[idxg.at[bg].at[jj].at[0]],
                                  gbuf.at[b], gsems[b]).start()

        def wait_gather(b):
            pltpu.make_async_copy(h_hbm.at[idxg.at[0].at[0].at[0]],
                                  gbuf.at[b], gsems[b]).wait()

        def scatter(bg, jj, b):
            pltpu.sync_copy(gbuf.at[b], acc.at[idxg.at[bg].at[jj].at[1]],
                            add=True)

        def init_main(ref):
            return pltpu.make_async_copy(ref.at[pl.ds(s * RPT, RPT)],
                                         acc.at[pl.ds(s * RPT, RPT)], zsem)

        def init_rem(ref):
            return pltpu.make_async_copy(ref.at[pl.ds(_NS * RPT, REM)],
                                         acc.at[pl.ds(_NS * RPT, REM)], zsem)

        load_group(0, 0)
        load_group(1, 1)

        @pl.when(c == 0)
        def _():
            init_main(h_hbm).start()

        @pl.when(c != 0)
        def _():
            init_main(z_hbm).start()

        @pl.when(jnp.logical_and(s == 0, c == 0))
        def _():
            init_rem(h_hbm).start()

        @pl.when(jnp.logical_and(s == 0, c != 0))
        def _():
            init_rem(z_hbm).start()

        wait_group(0)
        start_gather(0, 0, 0)
        start_gather(0, 1, 1)

        init_main(z_hbm).wait()

        @pl.when(s == 0)
        def _():
            init_rem(z_hbm).wait()

        plsc.subcore_barrier()

        for g in range(NGRP):
            bg = g & 1
            nbg = bg ^ 1

            @pl.loop(0, (GB - 2) // 2)
            def _(ii):
                jj0 = 2 * ii
                wait_gather(0)
                scatter(bg, jj0, 0)
                start_gather(bg, jj0 + 2, 0)
                wait_gather(1)
                scatter(bg, jj0 + 1, 1)
                start_gather(bg, jj0 + 3, 1)

            wait_gather(0)
            scatter(bg, GB - 2, 0)
            if g + 1 < NGRP:
                wait_group(nbg)
                start_gather(nbg, 0, 0)
            wait_gather(1)
            scatter(bg, GB - 1, 1)
            if g + 1 < NGRP:
                start_gather(nbg, 1, 1)
            if g + 2 < NGRP:
                load_group(g + 2, bg)

        plsc.subcore_barrier()

        pltpu.sync_copy(acc.at[pl.ds(s * RPT, RPT)],
                        out_hbm.at[c].at[pl.ds(s * RPT, RPT)])

        @pl.when(s == 0)
        def _():
            pltpu.sync_copy(acc.at[pl.ds(_NS * RPT, REM)],
                            out_hbm.at[c].at[pl.ds(_NS * RPT, REM)])

    return agg_kernel(h, eidx, zeros)


def _tc_layer(agg, batch3d, wa, ba, ga, bea, wb, bb, gb, beb, head=None):
    _, N, D = agg.shape
    H = wa.shape[1]
    G = 128
    R = 10000
    NB = N // R
    assert NB * R == N

    def body(*refs):
        if head is None:
            (a_ref, b_ref, wa_ref, ba_ref, ga_ref, bea_ref,
             wb_ref, bb_ref, gb_ref, beb_ref,
             out_ref, pooled_ref,
             zbuf, s1, s2, t1, t2, sc1, sh1, sc2, sh2, pacc) = refs
        else:
            (a_ref, b_ref, wa_ref, ba_ref, ga_ref, bea_ref,
             wb_ref, bb_ref, gb_ref, beb_ref,
             p1_ref, p2_ref, w1a_ref, w1b_ref, w1c_ref, b1_ref,
             w2_ref, b2_ref,
             hout_ref,
             zbuf, s1, s2, t1, t2, sc1, sh1, sc2, sh2, pacc) = refs
        p = pl.program_id(0)
        j = pl.program_id(1)

        @pl.when(p == 0)
        def _():
            a = a_ref[0] + a_ref[1]
            z = jnp.dot(a, wa_ref[...],
                        preferred_element_type=jnp.float32) + ba_ref[...]
            zbuf[pl.ds(j * R, R), :] = z
            cs = jnp.sum(z, axis=0, keepdims=True)
            cq = jnp.sum(z * z, axis=0, keepdims=True)

            @pl.when(j == 0)
            def _():
                s1[...] = cs
                s2[...] = cq

            @pl.when(j > 0)
            def _():
                s1[...] += cs
                s2[...] += cq

        @pl.when(p == 1)
        def _():
            @pl.when(j == 0)
            def _():
                mean = s1[...] * (1.0 / N)
                var = s2[...] * (1.0 / N) - mean * mean
                sc = ga_ref[...] * lax.rsqrt(var + _EPS)
                sc1[...] = sc
                sh1[...] = bea_ref[...] - mean * sc

            z = zbuf[pl.ds(j * R, R), :]
            y = jnp.maximum(z * sc1[...] + sh1[...], 0.0)
            w = jnp.dot(y, wb_ref[...],
                        preferred_element_type=jnp.float32) + bb_ref[...]
            zbuf[pl.ds(j * R, R), :] = w
            cs = jnp.sum(w, axis=0, keepdims=True)
            cq = jnp.sum(w * w, axis=0, keepdims=True)

            @pl.when(j == 0)
            def _():
                t1[...] = cs
                t2[...] = cq

            @pl.when(j > 0)
            def _():
                t1[...] += cs
                t2[...] += cq

        @pl.when(p == 2)
        def _():
            @pl.when(j == 0)
            def _():
                mean = t1[...] * (1.0 / N)
                var = t2[...] * (1.0 / N) - mean * mean
                sc = gb_ref[...] * lax.rsqrt(var + _EPS)
                sc2[...] = sc
                sh2[...] = beb_ref[...] - mean * sc

            w = zbuf[pl.ds(j * R, R), :]
            y2 = jnp.maximum(w * sc2[...] + sh2[...], 0.0)
            if head is None:
                out_ref[...] = y2
            seg = b_ref[0]
            gi = lax.broadcasted_iota(jnp.int32, (G, R), 0)
            oh = (seg == gi).astype(jnp.float32)
            cp = jnp.dot(oh, y2, preferred_element_type=jnp.float32)

            @pl.when(j == 0)
            def _():
                pacc[...] = cp

            @pl.when(j > 0)
            def _():
                pacc[...] += cp

            @pl.when(j == NB - 1)
            def _():
                if head is None:
                    pooled_ref[...] = pacc[...]
                else:
                    z1 = (jnp.dot(p1_ref[...], w1a_ref[...],
                                  preferred_element_type=jnp.float32)
                          + jnp.dot(p2_ref[...], w1b_ref[...],
                                    preferred_element_type=jnp.float32)
                          + jnp.dot(pacc[...], w1c_ref[...],
                                    preferred_element_type=jnp.float32)
                          + b1_ref[...])
                    y1 = jnp.maximum(z1, 0.0)
                    hout_ref[...] = jnp.dot(
                        y1, w2_ref[...],
                        preferred_element_type=jnp.float32) + b2_ref[...]

    agg_p0 = pl.BlockSpec((2, R, D),
                          lambda p, j: (0, jnp.where(p == 0, j, 0), 0))
    b_p2 = pl.BlockSpec((1, 1, R),
                        lambda p, j: (jnp.where(p == 2, j, 0), 0, 0))
    full_spec = pl.BlockSpec((D, H), lambda p, j: (0, 0))
    vec_spec = pl.BlockSpec((1, H), lambda p, j: (0, 0))
    gh_spec = pl.BlockSpec((G, H), lambda p, j: (0, 0))

    in_specs = [agg_p0, b_p2,
                full_spec, vec_spec, vec_spec, vec_spec,
                full_spec, vec_spec, vec_spec, vec_spec]
    inputs = [agg, batch3d, wa, ba, ga, bea, wb, bb, gb, beb]
    if head is None:
        out_specs = [pl.BlockSpec((R, H),
                                  lambda p, j: (jnp.where(p == 2, j, 0), 0)),
                     gh_spec]
        out_shape = [jax.ShapeDtypeStruct((N, H), jnp.float32),
                     jax.ShapeDtypeStruct((G, H), jnp.float32)]
    else:
        pld1, pld2, w1a, w1b, w1c, b1, w2p, b2p = head
        OP = w2p.shape[1]
        in_specs += [gh_spec, gh_spec, full_spec, full_spec, full_spec,
                     vec_spec, pl.BlockSpec((H, OP), lambda p, j: (0, 0)),
                     pl.BlockSpec((1, OP), lambda p, j: (0, 0))]
        inputs += [pld1, pld2, w1a, w1b, w1c, b1, w2p, b2p]
        out_specs = pl.BlockSpec((G, OP), lambda p, j: (0, 0))
        out_shape = jax.ShapeDtypeStruct((G, OP), jnp.float32)

    return pl.pallas_call(
        body,
        grid=(3, NB),
        in_specs=in_specs,
        out_specs=out_specs,
        out_shape=out_shape,
        scratch_shapes=[
            pltpu.VMEM((N, H), jnp.float32),
            pltpu.VMEM((1, H), jnp.float32), pltpu.VMEM((1, H), jnp.float32),
            pltpu.VMEM((1, H), jnp.float32), pltpu.VMEM((1, H), jnp.float32),
            pltpu.VMEM((1, H), jnp.float32), pltpu.VMEM((1, H), jnp.float32),
            pltpu.VMEM((1, H), jnp.float32), pltpu.VMEM((1, H), jnp.float32),
            pltpu.VMEM((G, H), jnp.float32),
        ],
    )(*inputs)


@jax.jit
def kernel(x, edge_index, batch, params):
    src = edge_index[0]
    dst = edge_index[1]
    H = params["W0a"].shape[1]
    C = params["W_lin2"].shape[1]

    batch3d = batch.reshape(-1, 1, 10000)
    w1 = params["W_lin1"]
    w2p = jnp.pad(params["W_lin2"], ((0, 0), (0, 128 - C)))
    b2p = jnp.pad(params["b_lin2"], (0, 128 - C)).reshape(1, 128)

    def layer_params(l):
        return (params[f"W{l}a"], params[f"b{l}a"].reshape(1, H),
                params[f"g{l}a"].reshape(1, H), params[f"be{l}a"].reshape(1, H),
                params[f"W{l}b"], params[f"b{l}b"].reshape(1, H),
                params[f"g{l}b"].reshape(1, H), params[f"be{l}b"].reshape(1, H))

    h = x
    agg = _sc_edge_agg(h, src, dst)
    h, pld1 = _tc_layer(agg, batch3d, *layer_params(0))
    agg = _sc_edge_agg(h, src, dst)
    h, pld2 = _tc_layer(agg, batch3d, *layer_params(1))
    agg = _sc_edge_agg(h, src, dst)
    out = _tc_layer(agg, batch3d, *layer_params(2),
                    head=(pld1, pld2, w1[0:H], w1[H:2 * H], w1[2 * H:3 * H],
                          params["b_lin1"].reshape(1, H), w2p, b2p))
    return out[:, :C]
